# baseline (speedup 1.0000x reference)
import functools

import jax
import jax.numpy as jnp
from jax.experimental import pallas as pl

_STRIDES = [4, 4, 4, 4]
_NSAMPLE = [16, 16, 16, 16]


def _fps_body(pxyz_ref, out_ref):
    nb, n = pxyz_ref.shape[1], pxyz_ref.shape[2]
    m = out_ref.shape[0]
    px = pxyz_ref[0]
    py = pxyz_ref[1]
    pz = pxyz_ref[2]
    iota = jax.lax.broadcasted_iota(jnp.int32, (nb, n), 1)
    out_ref[0:1] = jnp.zeros((1, nb, 8), jnp.int32)

    def body(i, carry):
        dist, selx, sely, selz = carry
        dx = px - selx
        dy = py - sely
        dz = pz - selz
        d = dx * dx + dy * dy + dz * dz
        dist = jnp.minimum(dist, d)
        mx = jnp.max(dist, axis=1, keepdims=True)
        idx = jnp.min(jnp.where(dist == mx, iota, n), axis=1, keepdims=True)
        out_ref[pl.ds(i, 1)] = jnp.broadcast_to(idx, (nb, 8))[None]
        sel = iota == idx
        selx = jnp.sum(jnp.where(sel, px, 0.0), axis=1, keepdims=True)
        sely = jnp.sum(jnp.where(sel, py, 0.0), axis=1, keepdims=True)
        selz = jnp.sum(jnp.where(sel, pz, 0.0), axis=1, keepdims=True)
        return dist, selx, sely, selz

    dist0 = jnp.full((nb, n), jnp.inf, dtype=jnp.float32)
    jax.lax.fori_loop(
        1, m, body, (dist0, px[:, 0:1], py[:, 0:1], pz[:, 0:1]))


def _fps_batched(pts, m):
    nb, n, _ = pts.shape
    pxyz = pts.transpose(2, 0, 1)
    out = pl.pallas_call(
        _fps_body,
        out_shape=jax.ShapeDtypeStruct((m, nb, 8), jnp.int32),
    )(pxyz)
    return out[:, :, 0].transpose(1, 0)


def _topk_body(k, with_w, qx, qy, qz, rx, ry, rz, *outs):
    m = qx.shape[1]
    n = rx.shape[2]
    dx = qx[0] - rx[0]
    dy = qy[0] - ry[0]
    dz = qz[0] - rz[0]
    d = dx * dx + dy * dy + dz * dz
    iota = jax.lax.broadcasted_iota(jnp.int32, (m, n), 1)
    cols = []
    dds = []
    for _ in range(k):
        mn = jnp.min(d, axis=1, keepdims=True)
        idx = jnp.min(jnp.where(d == mn, iota, n), axis=1, keepdims=True)
        cols.append(idx)
        dds.append(mn)
        d = jnp.where(iota == idx, jnp.inf, d)
    ki = jnp.concatenate(cols, axis=1)
    outs[0][0] = ki
    if with_w:
        kd = jnp.concatenate(dds, axis=1)
        dist = jnp.sqrt(jnp.maximum(kd, 0.0))
        ww = 1.0 / (dist + 1e-8)
        ww = ww / ww.sum(1, keepdims=True)
        outs[1][0] = ww


def _knn_batched(q, ref, k, with_w=False):
    nb, m, _ = q.shape
    n = ref.shape[1]
    qt = q.transpose(2, 0, 1)[..., None]
    rt = ref.transpose(2, 0, 1)[:, :, None, :]
    out_shape = [jax.ShapeDtypeStruct((nb, m, k), jnp.int32)]
    out_specs = [pl.BlockSpec((1, m, k), lambda b: (b, 0, 0))]
    if with_w:
        out_shape.append(jax.ShapeDtypeStruct((nb, m, k), jnp.float32))
        out_specs.append(pl.BlockSpec((1, m, k), lambda b: (b, 0, 0)))
    res = pl.pallas_call(
        functools.partial(_topk_body, k, with_w),
        grid=(nb,),
        in_specs=[pl.BlockSpec((1, m, 1), lambda b: (b, 0, 0))] * 3
        + [pl.BlockSpec((1, 1, n), lambda b: (b, 0, 0))] * 3,
        out_specs=out_specs,
        out_shape=out_shape,
    )(qt[0], qt[1], qt[2], rt[0], rt[1], rt[2])
    return res if with_w else res[0]


def _geometry(p0, o):
    nb = o.shape[0]
    seg = p0.shape[0] // nb
    levels = []
    cur_p = p0
    cur_n = seg
    cur_starts = (o - seg).astype(jnp.int32)
    for st, ns in zip(_STRIDES, _NSAMPLE):
        m = cur_n // st
        segs = cur_p.reshape(nb, cur_n, 3)
        fi = _fps_batched(segs, m)
        samp = (fi + cur_starts[:, None]).reshape(-1)
        q = jnp.take_along_axis(segs, fi[..., None], axis=1)
        ki = _knn_batched(q, segs, ns)
        nbr = (ki + cur_starts[:, None, None]).reshape(-1, ns)
        new_p = cur_p[samp]
        rel = cur_p[nbr] - new_p[:, None, :]
        new_offs = [(b + 1) * m for b in range(nb)]
        levels.append({"samp": samp, "nbr": nbr, "rel": rel.astype(jnp.float32), "offs": new_offs})
        cur_p = new_p
        cur_n = m
        cur_starts = jnp.arange(nb, dtype=jnp.int32) * m
    return levels


def _interp_geom(p_fine, offs_fine, p_coarse, offs_coarse):
    nb = len(offs_fine)
    mf = offs_fine[0]
    nc = offs_coarse[0]
    q = p_fine.reshape(nb, mf, 3)
    ref = p_coarse.reshape(nb, nc, 3)
    ki, ww = _knn_batched(q, ref, 3, with_w=True)
    starts = jnp.arange(nb, dtype=jnp.int32)[:, None, None] * nc
    ii = (ki + starts).reshape(-1, 3)
    return ii, ww.reshape(-1, 3)


def _bn(x, g, b):
    ax = tuple(range(x.ndim - 1))
    m = x.mean(ax)
    v = x.var(ax)
    return (x - m) / jnp.sqrt(v + 1e-5) * g + b


def _enc1_kernel(x0_ref, w_ref, g_ref, b_ref, out_ref):
    h = jnp.dot(x0_ref[...], w_ref[...], preferred_element_type=jnp.float32)
    m = h.mean(axis=0, keepdims=True)
    v = ((h - m) ** 2).mean(axis=0, keepdims=True)
    hn = (h - m) / jnp.sqrt(v + 1e-5) * g_ref[...] + b_ref[...]
    out_ref[...] = jnp.maximum(hn, 0.0)


def _enc1(x0, W, g, b):
    n = x0.shape[0]
    co = W.shape[1]
    return pl.pallas_call(
        _enc1_kernel,
        out_shape=jax.ShapeDtypeStruct((n, co), jnp.float32),
    )(x0, W, g.reshape(1, co), b.reshape(1, co))


def _enc_stats_body(rel_ref, gf_ref, wr_ref, wf_ref, sum_ref, ssq_ref):
    h = jnp.dot(rel_ref[...], wr_ref[...], preferred_element_type=jnp.float32)
    h = h + jnp.dot(gf_ref[...], wf_ref[...], preferred_element_type=jnp.float32)
    s = h.sum(axis=0, keepdims=True)
    ss = (h * h).sum(axis=0, keepdims=True)

    @pl.when(pl.program_id(0) == 0)
    def _init():
        sum_ref[...] = s
        ssq_ref[...] = ss

    @pl.when(pl.program_id(0) > 0)
    def _acc():
        sum_ref[...] += s
        ssq_ref[...] += ss


def _enc_norm_body(nn, binv, rel_ref, gf_ref, wr_ref, wf_ref, g_ref, b_ref,
                   sum_ref, ssq_ref, out_ref):
    h = jnp.dot(rel_ref[...], wr_ref[...], preferred_element_type=jnp.float32)
    h = h + jnp.dot(gf_ref[...], wf_ref[...], preferred_element_type=jnp.float32)
    mu = sum_ref[...] * binv
    var = ssq_ref[...] * binv - mu * mu
    y = (h - mu) / jnp.sqrt(var + 1e-5) * g_ref[...] + b_ref[...]
    y = jnp.maximum(y, 0.0)
    mb = y.shape[0] // nn
    out_ref[...] = y.reshape(mb, nn, y.shape[1]).max(axis=1)


def _enc_level(rel_flat, gfeat, W, g, b, nn):
    B, C = gfeat.shape
    Co = W.shape[1]
    Wr = W[:3]
    Wf = W[3:]
    m = B // nn
    nblk = max(1, B // 8192)
    Bb = B // nblk
    mb = m // nblk
    sums, ssqs = pl.pallas_call(
        _enc_stats_body,
        grid=(nblk,),
        in_specs=[
            pl.BlockSpec((Bb, 3), lambda i: (i, 0)),
            pl.BlockSpec((Bb, C), lambda i: (i, 0)),
            pl.BlockSpec((3, Co), lambda i: (0, 0)),
            pl.BlockSpec((C, Co), lambda i: (0, 0)),
        ],
        out_specs=[
            pl.BlockSpec((1, Co), lambda i: (0, 0)),
            pl.BlockSpec((1, Co), lambda i: (0, 0)),
        ],
        out_shape=[
            jax.ShapeDtypeStruct((1, Co), jnp.float32),
            jax.ShapeDtypeStruct((1, Co), jnp.float32),
        ],
    )(rel_flat, gfeat, Wr, Wf)
    out = pl.pallas_call(
        functools.partial(_enc_norm_body, nn, 1.0 / B),
        grid=(nblk,),
        in_specs=[
            pl.BlockSpec((Bb, 3), lambda i: (i, 0)),
            pl.BlockSpec((Bb, C), lambda i: (i, 0)),
            pl.BlockSpec((3, Co), lambda i: (0, 0)),
            pl.BlockSpec((C, Co), lambda i: (0, 0)),
            pl.BlockSpec((1, Co), lambda i: (0, 0)),
            pl.BlockSpec((1, Co), lambda i: (0, 0)),
            pl.BlockSpec((1, Co), lambda i: (0, 0)),
            pl.BlockSpec((1, Co), lambda i: (0, 0)),
        ],
        out_specs=pl.BlockSpec((mb, Co), lambda i: (i, 0)),
        out_shape=jax.ShapeDtypeStruct((m, Co), jnp.float32),
    )(rel_flat, gfeat, Wr, Wf, g.reshape(1, Co), b.reshape(1, Co), sums, ssqs)
    return out


def _dec5_body(x5_ref, w2_ref, b2_ref, w1_ref, b1_ref, g_ref, bb_ref, out_ref):
    x5 = x5_ref[...]
    n, c = x5.shape
    x3d = x5.reshape(8, n // 8, c)
    mean = x3d.mean(axis=1)
    gf = jnp.dot(mean, w2_ref[...], preferred_element_type=jnp.float32) + b2_ref[...]
    gf = jnp.maximum(gf, 0.0)
    gfb = jnp.broadcast_to(gf[:, None, :], (8, n // 8, gf.shape[1]))
    xc = jnp.concatenate([x3d, gfb], axis=2).reshape(n, c + gf.shape[1])
    h = jnp.dot(xc, w1_ref[...], preferred_element_type=jnp.float32) + b1_ref[...]
    mu = h.mean(axis=0, keepdims=True)
    var = ((h - mu) ** 2).mean(axis=0, keepdims=True)
    y = (h - mu) / jnp.sqrt(var + 1e-5) * g_ref[...] + bb_ref[...]
    out_ref[...] = jnp.maximum(y, 0.0)


def _dec5(x5, P):
    n, c = x5.shape
    co = P["dec5_l1_W"].shape[1]
    return pl.pallas_call(
        _dec5_body,
        out_shape=jax.ShapeDtypeStruct((n, co), jnp.float32),
    )(x5, P["dec5_l2_W"], P["dec5_l2_b"].reshape(1, -1), P["dec5_l1_W"],
      P["dec5_l1_b"].reshape(1, -1), P["dec5_l1_g"].reshape(1, -1),
      P["dec5_l1_bb"].reshape(1, -1))


def _lin_bn_relu_body(x_ref, w_ref, b_ref, g_ref, bb_ref, out_ref):
    h = jnp.dot(x_ref[...], w_ref[...], preferred_element_type=jnp.float32) + b_ref[...]
    mu = h.mean(axis=0, keepdims=True)
    var = ((h - mu) ** 2).mean(axis=0, keepdims=True)
    y = (h - mu) / jnp.sqrt(var + 1e-5) * g_ref[...] + bb_ref[...]
    out_ref[...] = jnp.maximum(y, 0.0)


def _lin_bn_relu(xin, W, b, g, bb):
    n = xin.shape[0]
    co = W.shape[1]
    return pl.pallas_call(
        _lin_bn_relu_body,
        out_shape=jax.ShapeDtypeStruct((n, co), jnp.float32),
    )(xin, W, b.reshape(1, co), g.reshape(1, co), bb.reshape(1, co))


def _dec_combine_body(x_ref, w_ref, b_ref, sum_ref, ssq_ref):
    h = jnp.dot(x_ref[...], w_ref[...], preferred_element_type=jnp.float32) + b_ref[...]
    s = h.sum(axis=0, keepdims=True)
    ss = (h * h).sum(axis=0, keepdims=True)

    @pl.when(pl.program_id(0) == 0)
    def _init():
        sum_ref[...] = s
        ssq_ref[...] = ss

    @pl.when(pl.program_id(0) > 0)
    def _acc():
        sum_ref[...] += s
        ssq_ref[...] += ss


def _dec_combine_norm_body(binv, x_ref, w_ref, b_ref, g_ref, bb_ref, gb_ref,
                           ww_ref, sum_ref, ssq_ref, out_ref):
    h = jnp.dot(x_ref[...], w_ref[...], preferred_element_type=jnp.float32) + b_ref[...]
    mu = sum_ref[...] * binv
    var = ssq_ref[...] * binv - mu * mu
    a = (h - mu) / jnp.sqrt(var + 1e-5) * g_ref[...] + bb_ref[...]
    a = jnp.maximum(a, 0.0)
    co = h.shape[1]
    gb = gb_ref[...]
    ww = ww_ref[...]
    wsum = (gb[:, :co] * ww[:, 0:1] + gb[:, co:2 * co] * ww[:, 1:2]
            + gb[:, 2 * co:] * ww[:, 2:3])
    out_ref[...] = a + wsum


def _dec_level(skip, W1, b1, g1, bb1, gb, ww):
    n, ci = skip.shape
    co = W1.shape[1]
    nblk = max(1, (n * max(ci, co)) // (4096 * 128))
    while n % nblk:
        nblk -= 1
    nb_ = n // nblk
    specs = [
        pl.BlockSpec((nb_, ci), lambda i: (i, 0)),
        pl.BlockSpec((ci, co), lambda i: (0, 0)),
        pl.BlockSpec((1, co), lambda i: (0, 0)),
        pl.BlockSpec((1, co), lambda i: (0, 0)),
        pl.BlockSpec((1, co), lambda i: (0, 0)),
        pl.BlockSpec((nb_, 3 * co), lambda i: (i, 0)),
        pl.BlockSpec((nb_, 3), lambda i: (i, 0)),
    ]
    stat_spec = [
        pl.BlockSpec((1, co), lambda i: (0, 0)),
        pl.BlockSpec((1, co), lambda i: (0, 0)),
    ]
    args = (skip, W1, b1.reshape(1, co), g1.reshape(1, co), bb1.reshape(1, co),
            gb, ww)
    sums, ssqs = pl.pallas_call(
        _dec_combine_body,
        grid=(nblk,),
        in_specs=specs[:3],
        out_specs=stat_spec,
        out_shape=[jax.ShapeDtypeStruct((1, co), jnp.float32)] * 2,
    )(*args[:3])
    return pl.pallas_call(
        functools.partial(_dec_combine_norm_body, 1.0 / n),
        grid=(nblk,),
        in_specs=specs + stat_spec,
        out_specs=pl.BlockSpec((nb_, co), lambda i: (i, 0)),
        out_shape=jax.ShapeDtypeStruct((n, co), jnp.float32),
    )(*args, sums, ssqs)


def _forward(x0, P, geom, interp, offs_list):
    x1 = _enc1(x0, P["enc1_W"], P["enc1_g"], P["enc1_b"])
    feats = [x1]
    cur = x1
    for li in range(2, 6):
        g = geom[li - 2]
        rel = jnp.asarray(g["rel"])
        nbr = jnp.asarray(g["nbr"])
        ns = nbr.shape[1]
        gfeat = cur[nbr.reshape(-1)]
        rel_flat = rel.reshape(-1, 3)
        cur = _enc_level(rel_flat, gfeat, P["enc%d_W" % li],
                         P["enc%d_g" % li], P["enc%d_b" % li], ns)
        feats.append(cur)
    x1, x2, x3, x4, x5 = feats
    up = _dec5(x5, P)
    skips = [x4, x3, x2, x1]
    for di, skip in zip([4, 3, 2, 1], skips):
        ii, ww = interp[di]
        bfeat = _lin_bn_relu(up, P["dec%d_l2_W" % di], P["dec%d_l2_b" % di],
                             P["dec%d_l2_g" % di], P["dec%d_l2_bb" % di])
        gb = bfeat[ii.reshape(-1)].reshape(ii.shape[0], -1)
        up = _dec_level(skip, P["dec%d_l1_W" % di], P["dec%d_l1_b" % di],
                        P["dec%d_l1_g" % di], P["dec%d_l1_bb" % di],
                        gb, ww)
    return up


def kernel(p, x, o, params):
    nb = o.shape[0]
    seg = p.shape[0] // nb
    geom = _geometry(p, o)
    offs0 = [(b + 1) * seg for b in range(nb)]
    offs_list = [offs0] + [g["offs"] for g in geom]
    p_levels = [p]
    cur = p
    for g in geom:
        cur = cur[g["samp"]]
        p_levels.append(cur)
    interp = {}
    for di, (fi, ci) in zip([4, 3, 2, 1], [(3, 4), (2, 3), (1, 2), (0, 1)]):
        ii, ww = _interp_geom(p_levels[fi], offs_list[fi], p_levels[ci], offs_list[ci])
        interp[di] = (ii, ww)
    x0 = jnp.concatenate([p, x], 1)
    return _forward(x0, params, geom, interp, offs_list)


# speedup vs baseline: 1.0141x; 1.0141x over previous
import functools

import jax
import jax.numpy as jnp
from jax import lax
from jax.experimental import pallas as pl
from jax.experimental.pallas import tpu as pltpu
from jax.experimental.pallas import tpu_sc as plsc

_STRIDES = [4, 4, 4, 4]
_NSAMPLE = [16, 16, 16, 16]


def _fps_body(pxyz_ref, out_ref):
    nb, n = pxyz_ref.shape[1], pxyz_ref.shape[2]
    m = out_ref.shape[0]
    px = pxyz_ref[0]
    py = pxyz_ref[1]
    pz = pxyz_ref[2]
    iota = jax.lax.broadcasted_iota(jnp.int32, (nb, n), 1)
    out_ref[0:1] = jnp.zeros((1, nb, 8), jnp.int32)

    def body(i, carry):
        dist, selx, sely, selz = carry
        dx = px - selx
        dy = py - sely
        dz = pz - selz
        d = dx * dx + dy * dy + dz * dz
        dist = jnp.minimum(dist, d)
        mx = jnp.max(dist, axis=1, keepdims=True)
        idx = jnp.min(jnp.where(dist == mx, iota, n), axis=1, keepdims=True)
        out_ref[pl.ds(i, 1)] = jnp.broadcast_to(idx, (nb, 8))[None]
        sel = iota == idx
        selx = jnp.sum(jnp.where(sel, px, 0.0), axis=1, keepdims=True)
        sely = jnp.sum(jnp.where(sel, py, 0.0), axis=1, keepdims=True)
        selz = jnp.sum(jnp.where(sel, pz, 0.0), axis=1, keepdims=True)
        return dist, selx, sely, selz

    dist0 = jnp.full((nb, n), jnp.inf, dtype=jnp.float32)
    jax.lax.fori_loop(
        1, m, body, (dist0, px[:, 0:1], py[:, 0:1], pz[:, 0:1]))


def _fps_batched(pts, m):
    nb, n, _ = pts.shape
    pxyz = pts.transpose(2, 0, 1)
    out = pl.pallas_call(
        _fps_body,
        out_shape=jax.ShapeDtypeStruct((m, nb, 8), jnp.int32),
    )(pxyz)
    return out[:, :, 0].transpose(1, 0)


def _topk_body(k, with_w, qx, qy, qz, rx, ry, rz, *outs):
    m = qx.shape[1]
    n = rx.shape[2]
    dx = qx[0] - rx[0]
    dy = qy[0] - ry[0]
    dz = qz[0] - rz[0]
    d = dx * dx + dy * dy + dz * dz
    iota = jax.lax.broadcasted_iota(jnp.int32, (m, n), 1)
    cols = []
    dds = []
    for _ in range(k):
        mn = jnp.min(d, axis=1, keepdims=True)
        idx = jnp.min(jnp.where(d == mn, iota, n), axis=1, keepdims=True)
        cols.append(idx)
        dds.append(mn)
        d = jnp.where(iota == idx, jnp.inf, d)
    ki = jnp.concatenate(cols, axis=1)
    outs[0][0] = ki
    if with_w:
        kd = jnp.concatenate(dds, axis=1)
        dist = jnp.sqrt(jnp.maximum(kd, 0.0))
        ww = 1.0 / (dist + 1e-8)
        ww = ww / ww.sum(1, keepdims=True)
        outs[1][0] = ww


def _knn_batched(q, ref, k, with_w=False):
    nb, m, _ = q.shape
    n = ref.shape[1]
    qt = q.transpose(2, 0, 1)[..., None]
    rt = ref.transpose(2, 0, 1)[:, :, None, :]
    out_shape = [jax.ShapeDtypeStruct((nb, m, k), jnp.int32)]
    out_specs = [pl.BlockSpec((1, m, k), lambda b: (b, 0, 0))]
    if with_w:
        out_shape.append(jax.ShapeDtypeStruct((nb, m, k), jnp.float32))
        out_specs.append(pl.BlockSpec((1, m, k), lambda b: (b, 0, 0)))
    res = pl.pallas_call(
        functools.partial(_topk_body, k, with_w),
        grid=(nb,),
        in_specs=[pl.BlockSpec((1, m, 1), lambda b: (b, 0, 0))] * 3
        + [pl.BlockSpec((1, 1, n), lambda b: (b, 0, 0))] * 3,
        out_specs=out_specs,
        out_shape=out_shape,
    )(qt[0], qt[1], qt[2], rt[0], rt[1], rt[2])
    return res if with_w else res[0]


def _geometry(p0, o):
    nb = o.shape[0]
    seg = p0.shape[0] // nb
    levels = []
    cur_p = p0
    cur_n = seg
    cur_starts = (o - seg).astype(jnp.int32)
    for st, ns in zip(_STRIDES, _NSAMPLE):
        m = cur_n // st
        segs = cur_p.reshape(nb, cur_n, 3)
        fi = _fps_batched(segs, m)
        samp = (fi + cur_starts[:, None]).reshape(-1)
        q = jnp.take_along_axis(segs, fi[..., None], axis=1)
        ki = _knn_batched(q, segs, ns)
        nbr = (ki + cur_starts[:, None, None]).reshape(-1, ns)
        new_p = cur_p[samp]
        rel = cur_p[nbr] - new_p[:, None, :]
        new_offs = [(b + 1) * m for b in range(nb)]
        levels.append({"samp": samp, "nbr": nbr, "rel": rel.astype(jnp.float32), "offs": new_offs})
        cur_p = new_p
        cur_n = m
        cur_starts = jnp.arange(nb, dtype=jnp.int32) * m
    return levels


def _interp_geom(p_fine, offs_fine, p_coarse, offs_coarse):
    nb = len(offs_fine)
    mf = offs_fine[0]
    nc = offs_coarse[0]
    q = p_fine.reshape(nb, mf, 3)
    ref = p_coarse.reshape(nb, nc, 3)
    ki, ww = _knn_batched(q, ref, 3, with_w=True)
    starts = jnp.arange(nb, dtype=jnp.int32)[:, None, None] * nc
    ii = (ki + starts).reshape(-1, 3)
    return ii, ww.reshape(-1, 3)


def _sc_gather(table, idx):
    V, D = table.shape
    B = idx.shape[0]
    NW = 32
    b_per_w = B // NW
    if D % 128 or B % (8 * NW):
        return table[idx]

    mesh = plsc.VectorSubcoreMesh(core_axis_name="c", subcore_axis_name="s")

    @functools.partial(
        pl.kernel,
        mesh=mesh,
        out_type=jax.ShapeDtypeStruct((B, D), jnp.float32),
        scratch_types=[
            pltpu.VMEM((b_per_w,), jnp.int32),
            pltpu.VMEM((b_per_w, D), jnp.float32),
            pltpu.SemaphoreType.DMA,
        ],
    )
    def k(table_hbm, idx_hbm, out_hbm, idx_v, rows_v, sem):
        wid = lax.axis_index("s") * 2 + lax.axis_index("c")
        base = wid * b_per_w
        pltpu.sync_copy(idx_hbm.at[pl.ds(base, b_per_w)], idx_v)
        pltpu.async_copy(table_hbm.at[idx_v], rows_v, sem).wait()
        pltpu.sync_copy(rows_v, out_hbm.at[pl.ds(base, b_per_w)])

    return k(table, idx)


def _bn(x, g, b):
    ax = tuple(range(x.ndim - 1))
    m = x.mean(ax)
    v = x.var(ax)
    return (x - m) / jnp.sqrt(v + 1e-5) * g + b


def _enc1_kernel(x0_ref, w_ref, g_ref, b_ref, out_ref):
    h = jnp.dot(x0_ref[...], w_ref[...], preferred_element_type=jnp.float32)
    m = h.mean(axis=0, keepdims=True)
    v = ((h - m) ** 2).mean(axis=0, keepdims=True)
    hn = (h - m) / jnp.sqrt(v + 1e-5) * g_ref[...] + b_ref[...]
    out_ref[...] = jnp.maximum(hn, 0.0)


def _enc1(x0, W, g, b):
    n = x0.shape[0]
    co = W.shape[1]
    return pl.pallas_call(
        _enc1_kernel,
        out_shape=jax.ShapeDtypeStruct((n, co), jnp.float32),
    )(x0, W, g.reshape(1, co), b.reshape(1, co))


def _enc_stats_body(rel_ref, gf_ref, wr_ref, wf_ref, sum_ref, ssq_ref):
    h = jnp.dot(rel_ref[...], wr_ref[...], preferred_element_type=jnp.float32)
    h = h + jnp.dot(gf_ref[...], wf_ref[...], preferred_element_type=jnp.float32)
    s = h.sum(axis=0, keepdims=True)
    ss = (h * h).sum(axis=0, keepdims=True)

    @pl.when(pl.program_id(0) == 0)
    def _init():
        sum_ref[...] = s
        ssq_ref[...] = ss

    @pl.when(pl.program_id(0) > 0)
    def _acc():
        sum_ref[...] += s
        ssq_ref[...] += ss


def _enc_norm_body(nn, binv, rel_ref, gf_ref, wr_ref, wf_ref, g_ref, b_ref,
                   sum_ref, ssq_ref, out_ref):
    h = jnp.dot(rel_ref[...], wr_ref[...], preferred_element_type=jnp.float32)
    h = h + jnp.dot(gf_ref[...], wf_ref[...], preferred_element_type=jnp.float32)
    mu = sum_ref[...] * binv
    var = ssq_ref[...] * binv - mu * mu
    y = (h - mu) / jnp.sqrt(var + 1e-5) * g_ref[...] + b_ref[...]
    y = jnp.maximum(y, 0.0)
    mb = y.shape[0] // nn
    out_ref[...] = y.reshape(mb, nn, y.shape[1]).max(axis=1)


def _enc_level(rel_flat, gfeat, W, g, b, nn):
    B, C = gfeat.shape
    Co = W.shape[1]
    Wr = W[:3]
    Wf = W[3:]
    m = B // nn
    nblk = max(1, B // 8192)
    Bb = B // nblk
    mb = m // nblk
    sums, ssqs = pl.pallas_call(
        _enc_stats_body,
        grid=(nblk,),
        in_specs=[
            pl.BlockSpec((Bb, 3), lambda i: (i, 0)),
            pl.BlockSpec((Bb, C), lambda i: (i, 0)),
            pl.BlockSpec((3, Co), lambda i: (0, 0)),
            pl.BlockSpec((C, Co), lambda i: (0, 0)),
        ],
        out_specs=[
            pl.BlockSpec((1, Co), lambda i: (0, 0)),
            pl.BlockSpec((1, Co), lambda i: (0, 0)),
        ],
        out_shape=[
            jax.ShapeDtypeStruct((1, Co), jnp.float32),
            jax.ShapeDtypeStruct((1, Co), jnp.float32),
        ],
    )(rel_flat, gfeat, Wr, Wf)
    out = pl.pallas_call(
        functools.partial(_enc_norm_body, nn, 1.0 / B),
        grid=(nblk,),
        in_specs=[
            pl.BlockSpec((Bb, 3), lambda i: (i, 0)),
            pl.BlockSpec((Bb, C), lambda i: (i, 0)),
            pl.BlockSpec((3, Co), lambda i: (0, 0)),
            pl.BlockSpec((C, Co), lambda i: (0, 0)),
            pl.BlockSpec((1, Co), lambda i: (0, 0)),
            pl.BlockSpec((1, Co), lambda i: (0, 0)),
            pl.BlockSpec((1, Co), lambda i: (0, 0)),
            pl.BlockSpec((1, Co), lambda i: (0, 0)),
        ],
        out_specs=pl.BlockSpec((mb, Co), lambda i: (i, 0)),
        out_shape=jax.ShapeDtypeStruct((m, Co), jnp.float32),
    )(rel_flat, gfeat, Wr, Wf, g.reshape(1, Co), b.reshape(1, Co), sums, ssqs)
    return out


def _dec5_body(x5_ref, w2_ref, b2_ref, w1_ref, b1_ref, g_ref, bb_ref, out_ref):
    x5 = x5_ref[...]
    n, c = x5.shape
    x3d = x5.reshape(8, n // 8, c)
    mean = x3d.mean(axis=1)
    gf = jnp.dot(mean, w2_ref[...], preferred_element_type=jnp.float32) + b2_ref[...]
    gf = jnp.maximum(gf, 0.0)
    gfb = jnp.broadcast_to(gf[:, None, :], (8, n // 8, gf.shape[1]))
    xc = jnp.concatenate([x3d, gfb], axis=2).reshape(n, c + gf.shape[1])
    h = jnp.dot(xc, w1_ref[...], preferred_element_type=jnp.float32) + b1_ref[...]
    mu = h.mean(axis=0, keepdims=True)
    var = ((h - mu) ** 2).mean(axis=0, keepdims=True)
    y = (h - mu) / jnp.sqrt(var + 1e-5) * g_ref[...] + bb_ref[...]
    out_ref[...] = jnp.maximum(y, 0.0)


def _dec5(x5, P):
    n, c = x5.shape
    co = P["dec5_l1_W"].shape[1]
    return pl.pallas_call(
        _dec5_body,
        out_shape=jax.ShapeDtypeStruct((n, co), jnp.float32),
    )(x5, P["dec5_l2_W"], P["dec5_l2_b"].reshape(1, -1), P["dec5_l1_W"],
      P["dec5_l1_b"].reshape(1, -1), P["dec5_l1_g"].reshape(1, -1),
      P["dec5_l1_bb"].reshape(1, -1))


def _lin_bn_relu_body(x_ref, w_ref, b_ref, g_ref, bb_ref, out_ref):
    h = jnp.dot(x_ref[...], w_ref[...], preferred_element_type=jnp.float32) + b_ref[...]
    mu = h.mean(axis=0, keepdims=True)
    var = ((h - mu) ** 2).mean(axis=0, keepdims=True)
    y = (h - mu) / jnp.sqrt(var + 1e-5) * g_ref[...] + bb_ref[...]
    out_ref[...] = jnp.maximum(y, 0.0)


def _lin_bn_relu(xin, W, b, g, bb):
    n = xin.shape[0]
    co = W.shape[1]
    return pl.pallas_call(
        _lin_bn_relu_body,
        out_shape=jax.ShapeDtypeStruct((n, co), jnp.float32),
    )(xin, W, b.reshape(1, co), g.reshape(1, co), bb.reshape(1, co))


def _dec_combine_body(x_ref, w_ref, b_ref, sum_ref, ssq_ref):
    h = jnp.dot(x_ref[...], w_ref[...], preferred_element_type=jnp.float32) + b_ref[...]
    s = h.sum(axis=0, keepdims=True)
    ss = (h * h).sum(axis=0, keepdims=True)

    @pl.when(pl.program_id(0) == 0)
    def _init():
        sum_ref[...] = s
        ssq_ref[...] = ss

    @pl.when(pl.program_id(0) > 0)
    def _acc():
        sum_ref[...] += s
        ssq_ref[...] += ss


def _dec_combine_norm_body(binv, x_ref, w_ref, b_ref, g_ref, bb_ref, gb_ref,
                           ww_ref, sum_ref, ssq_ref, out_ref):
    h = jnp.dot(x_ref[...], w_ref[...], preferred_element_type=jnp.float32) + b_ref[...]
    mu = sum_ref[...] * binv
    var = ssq_ref[...] * binv - mu * mu
    a = (h - mu) / jnp.sqrt(var + 1e-5) * g_ref[...] + bb_ref[...]
    a = jnp.maximum(a, 0.0)
    co = h.shape[1]
    gb = gb_ref[...]
    ww = ww_ref[...]
    wsum = (gb[:, :co] * ww[:, 0:1] + gb[:, co:2 * co] * ww[:, 1:2]
            + gb[:, 2 * co:] * ww[:, 2:3])
    out_ref[...] = a + wsum


def _dec_level(skip, W1, b1, g1, bb1, gb, ww):
    n, ci = skip.shape
    co = W1.shape[1]
    nblk = max(1, (n * max(ci, co)) // (4096 * 128))
    while n % nblk:
        nblk -= 1
    nb_ = n // nblk
    specs = [
        pl.BlockSpec((nb_, ci), lambda i: (i, 0)),
        pl.BlockSpec((ci, co), lambda i: (0, 0)),
        pl.BlockSpec((1, co), lambda i: (0, 0)),
        pl.BlockSpec((1, co), lambda i: (0, 0)),
        pl.BlockSpec((1, co), lambda i: (0, 0)),
        pl.BlockSpec((nb_, 3 * co), lambda i: (i, 0)),
        pl.BlockSpec((nb_, 3), lambda i: (i, 0)),
    ]
    stat_spec = [
        pl.BlockSpec((1, co), lambda i: (0, 0)),
        pl.BlockSpec((1, co), lambda i: (0, 0)),
    ]
    args = (skip, W1, b1.reshape(1, co), g1.reshape(1, co), bb1.reshape(1, co),
            gb, ww)
    sums, ssqs = pl.pallas_call(
        _dec_combine_body,
        grid=(nblk,),
        in_specs=specs[:3],
        out_specs=stat_spec,
        out_shape=[jax.ShapeDtypeStruct((1, co), jnp.float32)] * 2,
    )(*args[:3])
    return pl.pallas_call(
        functools.partial(_dec_combine_norm_body, 1.0 / n),
        grid=(nblk,),
        in_specs=specs + stat_spec,
        out_specs=pl.BlockSpec((nb_, co), lambda i: (i, 0)),
        out_shape=jax.ShapeDtypeStruct((n, co), jnp.float32),
    )(*args, sums, ssqs)


def _forward(x0, P, geom, interp, offs_list):
    x1 = _enc1(x0, P["enc1_W"], P["enc1_g"], P["enc1_b"])
    feats = [x1]
    cur = x1
    for li in range(2, 6):
        g = geom[li - 2]
        rel = jnp.asarray(g["rel"])
        nbr = jnp.asarray(g["nbr"])
        ns = nbr.shape[1]
        gfeat = _sc_gather(cur, nbr.reshape(-1))
        rel_flat = rel.reshape(-1, 3)
        cur = _enc_level(rel_flat, gfeat, P["enc%d_W" % li],
                         P["enc%d_g" % li], P["enc%d_b" % li], ns)
        feats.append(cur)
    x1, x2, x3, x4, x5 = feats
    up = _dec5(x5, P)
    skips = [x4, x3, x2, x1]
    for di, skip in zip([4, 3, 2, 1], skips):
        ii, ww = interp[di]
        bfeat = _lin_bn_relu(up, P["dec%d_l2_W" % di], P["dec%d_l2_b" % di],
                             P["dec%d_l2_g" % di], P["dec%d_l2_bb" % di])
        gb = _sc_gather(bfeat, ii.reshape(-1)).reshape(ii.shape[0], -1)
        up = _dec_level(skip, P["dec%d_l1_W" % di], P["dec%d_l1_b" % di],
                        P["dec%d_l1_g" % di], P["dec%d_l1_bb" % di],
                        gb, ww)
    return up


def kernel(p, x, o, params):
    nb = o.shape[0]
    seg = p.shape[0] // nb
    geom = _geometry(p, o)
    offs0 = [(b + 1) * seg for b in range(nb)]
    offs_list = [offs0] + [g["offs"] for g in geom]
    p_levels = [p]
    cur = p
    for g in geom:
        cur = cur[g["samp"]]
        p_levels.append(cur)
    interp = {}
    for di, (fi, ci) in zip([4, 3, 2, 1], [(3, 4), (2, 3), (1, 2), (0, 1)]):
        ii, ww = _interp_geom(p_levels[fi], offs_list[fi], p_levels[ci], offs_list[ci])
        interp[di] = (ii, ww)
    x0 = jnp.concatenate([p, x], 1)
    return _forward(x0, params, geom, interp, offs_list)


# speedup vs baseline: 1.0190x; 1.0048x over previous
import functools

import jax
import jax.numpy as jnp
from jax import lax
from jax.experimental import pallas as pl
from jax.experimental.pallas import tpu as pltpu
from jax.experimental.pallas import tpu_sc as plsc

_STRIDES = [4, 4, 4, 4]
_NSAMPLE = [16, 16, 16, 16]


def _fps_body(pxyz_ref, out_ref):
    nb, n = pxyz_ref.shape[1], pxyz_ref.shape[2]
    m = out_ref.shape[0]
    px = pxyz_ref[0]
    py = pxyz_ref[1]
    pz = pxyz_ref[2]
    bw = min(n, 128)
    nblk = n // bw
    lane = jax.lax.broadcasted_iota(jnp.int32, (nb, bw), 1)
    out_ref[0:1] = jnp.zeros((1, nb, 8), jnp.int32)

    def body(i, carry):
        dist, selx, sely, selz = carry
        dx = px - selx
        dy = py - sely
        dz = pz - selz
        d = dx * dx + dy * dy + dz * dz
        dist = jnp.minimum(dist, d)
        best = dist[:, 0:bw]
        bidx = lane
        for b in range(1, nblk):
            c = dist[:, b * bw:(b + 1) * bw]
            take = c > best
            best = jnp.where(take, c, best)
            bidx = jnp.where(take, lane + b * bw, bidx)
        mx = jnp.max(best, axis=1, keepdims=True)
        idx = jnp.min(jnp.where(best == mx, bidx, n), axis=1, keepdims=True)
        out_ref[pl.ds(i, 1)] = jnp.broadcast_to(idx, (nb, 8))[None]
        accx = jnp.zeros((nb, bw), jnp.float32)
        accy = accx
        accz = accx
        for b in range(nblk):
            sel = (lane + b * bw) == idx
            accx = accx + jnp.where(sel, px[:, b * bw:(b + 1) * bw], 0.0)
            accy = accy + jnp.where(sel, py[:, b * bw:(b + 1) * bw], 0.0)
            accz = accz + jnp.where(sel, pz[:, b * bw:(b + 1) * bw], 0.0)
        selx = jnp.sum(accx, axis=1, keepdims=True)
        sely = jnp.sum(accy, axis=1, keepdims=True)
        selz = jnp.sum(accz, axis=1, keepdims=True)
        return dist, selx, sely, selz

    dist0 = jnp.full((nb, n), jnp.inf, dtype=jnp.float32)
    jax.lax.fori_loop(
        1, m, body, (dist0, px[:, 0:1], py[:, 0:1], pz[:, 0:1]))


def _fps_batched(pts, m):
    nb, n, _ = pts.shape
    pxyz = pts.transpose(2, 0, 1)
    out = pl.pallas_call(
        _fps_body,
        out_shape=jax.ShapeDtypeStruct((m, nb, 8), jnp.int32),
    )(pxyz)
    return out[:, :, 0].transpose(1, 0)


def _topk_body(k, with_w, qx, qy, qz, rx, ry, rz, *outs):
    m = qx.shape[1]
    n = rx.shape[2]
    dx = qx[0] - rx[0]
    dy = qy[0] - ry[0]
    dz = qz[0] - rz[0]
    d = dx * dx + dy * dy + dz * dz
    iota = jax.lax.broadcasted_iota(jnp.int32, (m, n), 1)
    bw = min(n, 128)
    nblk = n // bw
    lane = jax.lax.broadcasted_iota(jnp.int32, (m, bw), 1)
    cols = []
    dds = []
    for _ in range(k):
        best = d[:, 0:bw]
        bidx = lane
        for b in range(1, nblk):
            c = d[:, b * bw:(b + 1) * bw]
            take = c < best
            best = jnp.where(take, c, best)
            bidx = jnp.where(take, lane + b * bw, bidx)
        mn = jnp.min(best, axis=1, keepdims=True)
        idx = jnp.min(jnp.where(best == mn, bidx, n), axis=1, keepdims=True)
        cols.append(idx)
        dds.append(mn)
        d = jnp.where(iota == idx, jnp.inf, d)
    ki = jnp.concatenate(cols, axis=1)
    outs[0][0] = ki
    if with_w:
        kd = jnp.concatenate(dds, axis=1)
        dist = jnp.sqrt(jnp.maximum(kd, 0.0))
        ww = 1.0 / (dist + 1e-8)
        ww = ww / ww.sum(1, keepdims=True)
        outs[1][0] = ww


def _knn_batched(q, ref, k, with_w=False):
    nb, m, _ = q.shape
    n = ref.shape[1]
    qt = q.transpose(2, 0, 1)[..., None]
    rt = ref.transpose(2, 0, 1)[:, :, None, :]
    out_shape = [jax.ShapeDtypeStruct((nb, m, k), jnp.int32)]
    out_specs = [pl.BlockSpec((1, m, k), lambda b: (b, 0, 0))]
    if with_w:
        out_shape.append(jax.ShapeDtypeStruct((nb, m, k), jnp.float32))
        out_specs.append(pl.BlockSpec((1, m, k), lambda b: (b, 0, 0)))
    res = pl.pallas_call(
        functools.partial(_topk_body, k, with_w),
        grid=(nb,),
        in_specs=[pl.BlockSpec((1, m, 1), lambda b: (b, 0, 0))] * 3
        + [pl.BlockSpec((1, 1, n), lambda b: (b, 0, 0))] * 3,
        out_specs=out_specs,
        out_shape=out_shape,
    )(qt[0], qt[1], qt[2], rt[0], rt[1], rt[2])
    return res if with_w else res[0]


def _geometry(p0, o):
    nb = o.shape[0]
    seg = p0.shape[0] // nb
    levels = []
    cur_p = p0
    cur_n = seg
    cur_starts = (o - seg).astype(jnp.int32)
    for st, ns in zip(_STRIDES, _NSAMPLE):
        m = cur_n // st
        segs = cur_p.reshape(nb, cur_n, 3)
        fi = _fps_batched(segs, m)
        samp = (fi + cur_starts[:, None]).reshape(-1)
        q = jnp.take_along_axis(segs, fi[..., None], axis=1)
        ki = _knn_batched(q, segs, ns)
        nbr = (ki + cur_starts[:, None, None]).reshape(-1, ns)
        new_p = cur_p[samp]
        rel = cur_p[nbr] - new_p[:, None, :]
        new_offs = [(b + 1) * m for b in range(nb)]
        levels.append({"samp": samp, "nbr": nbr, "rel": rel.astype(jnp.float32), "offs": new_offs})
        cur_p = new_p
        cur_n = m
        cur_starts = jnp.arange(nb, dtype=jnp.int32) * m
    return levels


def _interp_geom(p_fine, offs_fine, p_coarse, offs_coarse):
    nb = len(offs_fine)
    mf = offs_fine[0]
    nc = offs_coarse[0]
    q = p_fine.reshape(nb, mf, 3)
    ref = p_coarse.reshape(nb, nc, 3)
    ki, ww = _knn_batched(q, ref, 3, with_w=True)
    starts = jnp.arange(nb, dtype=jnp.int32)[:, None, None] * nc
    ii = (ki + starts).reshape(-1, 3)
    return ii, ww.reshape(-1, 3)


def _sc_gather(table, idx):
    V, D = table.shape
    B = idx.shape[0]
    NW = 32
    b_per_w = B // NW
    if D % 128 or B % (8 * NW):
        return table[idx]

    mesh = plsc.VectorSubcoreMesh(core_axis_name="c", subcore_axis_name="s")

    @functools.partial(
        pl.kernel,
        mesh=mesh,
        out_type=jax.ShapeDtypeStruct((B, D), jnp.float32),
        scratch_types=[
            pltpu.VMEM((b_per_w,), jnp.int32),
            pltpu.VMEM((b_per_w, D), jnp.float32),
            pltpu.SemaphoreType.DMA,
        ],
    )
    def k(table_hbm, idx_hbm, out_hbm, idx_v, rows_v, sem):
        wid = lax.axis_index("s") * 2 + lax.axis_index("c")
        base = wid * b_per_w
        pltpu.sync_copy(idx_hbm.at[pl.ds(base, b_per_w)], idx_v)
        pltpu.async_copy(table_hbm.at[idx_v], rows_v, sem).wait()
        pltpu.sync_copy(rows_v, out_hbm.at[pl.ds(base, b_per_w)])

    return k(table, idx)


def _bn(x, g, b):
    ax = tuple(range(x.ndim - 1))
    m = x.mean(ax)
    v = x.var(ax)
    return (x - m) / jnp.sqrt(v + 1e-5) * g + b


def _enc1_kernel(x0_ref, w_ref, g_ref, b_ref, out_ref):
    h = jnp.dot(x0_ref[...], w_ref[...], preferred_element_type=jnp.float32)
    m = h.mean(axis=0, keepdims=True)
    v = ((h - m) ** 2).mean(axis=0, keepdims=True)
    hn = (h - m) / jnp.sqrt(v + 1e-5) * g_ref[...] + b_ref[...]
    out_ref[...] = jnp.maximum(hn, 0.0)


def _enc1(x0, W, g, b):
    n = x0.shape[0]
    co = W.shape[1]
    return pl.pallas_call(
        _enc1_kernel,
        out_shape=jax.ShapeDtypeStruct((n, co), jnp.float32),
    )(x0, W, g.reshape(1, co), b.reshape(1, co))


def _enc_stats_body(rel_ref, gf_ref, wr_ref, wf_ref, sum_ref, ssq_ref):
    h = jnp.dot(rel_ref[...], wr_ref[...], preferred_element_type=jnp.float32)
    h = h + jnp.dot(gf_ref[...], wf_ref[...], preferred_element_type=jnp.float32)
    s = h.sum(axis=0, keepdims=True)
    ss = (h * h).sum(axis=0, keepdims=True)

    @pl.when(pl.program_id(0) == 0)
    def _init():
        sum_ref[...] = s
        ssq_ref[...] = ss

    @pl.when(pl.program_id(0) > 0)
    def _acc():
        sum_ref[...] += s
        ssq_ref[...] += ss


def _enc_norm_body(nn, binv, rel_ref, gf_ref, wr_ref, wf_ref, g_ref, b_ref,
                   sum_ref, ssq_ref, out_ref):
    h = jnp.dot(rel_ref[...], wr_ref[...], preferred_element_type=jnp.float32)
    h = h + jnp.dot(gf_ref[...], wf_ref[...], preferred_element_type=jnp.float32)
    mu = sum_ref[...] * binv
    var = ssq_ref[...] * binv - mu * mu
    y = (h - mu) / jnp.sqrt(var + 1e-5) * g_ref[...] + b_ref[...]
    y = jnp.maximum(y, 0.0)
    mb = y.shape[0] // nn
    out_ref[...] = y.reshape(mb, nn, y.shape[1]).max(axis=1)


def _enc_level(rel_flat, gfeat, W, g, b, nn):
    B, C = gfeat.shape
    Co = W.shape[1]
    Wr = W[:3]
    Wf = W[3:]
    m = B // nn
    nblk = max(1, B // 8192)
    Bb = B // nblk
    mb = m // nblk
    sums, ssqs = pl.pallas_call(
        _enc_stats_body,
        grid=(nblk,),
        in_specs=[
            pl.BlockSpec((Bb, 3), lambda i: (i, 0)),
            pl.BlockSpec((Bb, C), lambda i: (i, 0)),
            pl.BlockSpec((3, Co), lambda i: (0, 0)),
            pl.BlockSpec((C, Co), lambda i: (0, 0)),
        ],
        out_specs=[
            pl.BlockSpec((1, Co), lambda i: (0, 0)),
            pl.BlockSpec((1, Co), lambda i: (0, 0)),
        ],
        out_shape=[
            jax.ShapeDtypeStruct((1, Co), jnp.float32),
            jax.ShapeDtypeStruct((1, Co), jnp.float32),
        ],
    )(rel_flat, gfeat, Wr, Wf)
    out = pl.pallas_call(
        functools.partial(_enc_norm_body, nn, 1.0 / B),
        grid=(nblk,),
        in_specs=[
            pl.BlockSpec((Bb, 3), lambda i: (i, 0)),
            pl.BlockSpec((Bb, C), lambda i: (i, 0)),
            pl.BlockSpec((3, Co), lambda i: (0, 0)),
            pl.BlockSpec((C, Co), lambda i: (0, 0)),
            pl.BlockSpec((1, Co), lambda i: (0, 0)),
            pl.BlockSpec((1, Co), lambda i: (0, 0)),
            pl.BlockSpec((1, Co), lambda i: (0, 0)),
            pl.BlockSpec((1, Co), lambda i: (0, 0)),
        ],
        out_specs=pl.BlockSpec((mb, Co), lambda i: (i, 0)),
        out_shape=jax.ShapeDtypeStruct((m, Co), jnp.float32),
    )(rel_flat, gfeat, Wr, Wf, g.reshape(1, Co), b.reshape(1, Co), sums, ssqs)
    return out


def _dec5_body(x5_ref, w2_ref, b2_ref, w1_ref, b1_ref, g_ref, bb_ref, out_ref):
    x5 = x5_ref[...]
    n, c = x5.shape
    x3d = x5.reshape(8, n // 8, c)
    mean = x3d.mean(axis=1)
    gf = jnp.dot(mean, w2_ref[...], preferred_element_type=jnp.float32) + b2_ref[...]
    gf = jnp.maximum(gf, 0.0)
    gfb = jnp.broadcast_to(gf[:, None, :], (8, n // 8, gf.shape[1]))
    xc = jnp.concatenate([x3d, gfb], axis=2).reshape(n, c + gf.shape[1])
    h = jnp.dot(xc, w1_ref[...], preferred_element_type=jnp.float32) + b1_ref[...]
    mu = h.mean(axis=0, keepdims=True)
    var = ((h - mu) ** 2).mean(axis=0, keepdims=True)
    y = (h - mu) / jnp.sqrt(var + 1e-5) * g_ref[...] + bb_ref[...]
    out_ref[...] = jnp.maximum(y, 0.0)


def _dec5(x5, P):
    n, c = x5.shape
    co = P["dec5_l1_W"].shape[1]
    return pl.pallas_call(
        _dec5_body,
        out_shape=jax.ShapeDtypeStruct((n, co), jnp.float32),
    )(x5, P["dec5_l2_W"], P["dec5_l2_b"].reshape(1, -1), P["dec5_l1_W"],
      P["dec5_l1_b"].reshape(1, -1), P["dec5_l1_g"].reshape(1, -1),
      P["dec5_l1_bb"].reshape(1, -1))


def _lin_bn_relu_body(x_ref, w_ref, b_ref, g_ref, bb_ref, out_ref):
    h = jnp.dot(x_ref[...], w_ref[...], preferred_element_type=jnp.float32) + b_ref[...]
    mu = h.mean(axis=0, keepdims=True)
    var = ((h - mu) ** 2).mean(axis=0, keepdims=True)
    y = (h - mu) / jnp.sqrt(var + 1e-5) * g_ref[...] + bb_ref[...]
    out_ref[...] = jnp.maximum(y, 0.0)


def _lin_bn_relu(xin, W, b, g, bb):
    n = xin.shape[0]
    co = W.shape[1]
    return pl.pallas_call(
        _lin_bn_relu_body,
        out_shape=jax.ShapeDtypeStruct((n, co), jnp.float32),
    )(xin, W, b.reshape(1, co), g.reshape(1, co), bb.reshape(1, co))


def _dec_combine_body(x_ref, w_ref, b_ref, sum_ref, ssq_ref):
    h = jnp.dot(x_ref[...], w_ref[...], preferred_element_type=jnp.float32) + b_ref[...]
    s = h.sum(axis=0, keepdims=True)
    ss = (h * h).sum(axis=0, keepdims=True)

    @pl.when(pl.program_id(0) == 0)
    def _init():
        sum_ref[...] = s
        ssq_ref[...] = ss

    @pl.when(pl.program_id(0) > 0)
    def _acc():
        sum_ref[...] += s
        ssq_ref[...] += ss


def _dec_combine_norm_body(binv, x_ref, w_ref, b_ref, g_ref, bb_ref, gb_ref,
                           ww_ref, sum_ref, ssq_ref, out_ref):
    h = jnp.dot(x_ref[...], w_ref[...], preferred_element_type=jnp.float32) + b_ref[...]
    mu = sum_ref[...] * binv
    var = ssq_ref[...] * binv - mu * mu
    a = (h - mu) / jnp.sqrt(var + 1e-5) * g_ref[...] + bb_ref[...]
    a = jnp.maximum(a, 0.0)
    co = h.shape[1]
    gb = gb_ref[...]
    ww = ww_ref[...]
    wsum = (gb[:, :co] * ww[:, 0:1] + gb[:, co:2 * co] * ww[:, 1:2]
            + gb[:, 2 * co:] * ww[:, 2:3])
    out_ref[...] = a + wsum


def _dec_level(skip, W1, b1, g1, bb1, gb, ww):
    n, ci = skip.shape
    co = W1.shape[1]
    nblk = max(1, (n * max(ci, co)) // (4096 * 128))
    while n % nblk:
        nblk -= 1
    nb_ = n // nblk
    specs = [
        pl.BlockSpec((nb_, ci), lambda i: (i, 0)),
        pl.BlockSpec((ci, co), lambda i: (0, 0)),
        pl.BlockSpec((1, co), lambda i: (0, 0)),
        pl.BlockSpec((1, co), lambda i: (0, 0)),
        pl.BlockSpec((1, co), lambda i: (0, 0)),
        pl.BlockSpec((nb_, 3 * co), lambda i: (i, 0)),
        pl.BlockSpec((nb_, 3), lambda i: (i, 0)),
    ]
    stat_spec = [
        pl.BlockSpec((1, co), lambda i: (0, 0)),
        pl.BlockSpec((1, co), lambda i: (0, 0)),
    ]
    args = (skip, W1, b1.reshape(1, co), g1.reshape(1, co), bb1.reshape(1, co),
            gb, ww)
    sums, ssqs = pl.pallas_call(
        _dec_combine_body,
        grid=(nblk,),
        in_specs=specs[:3],
        out_specs=stat_spec,
        out_shape=[jax.ShapeDtypeStruct((1, co), jnp.float32)] * 2,
    )(*args[:3])
    return pl.pallas_call(
        functools.partial(_dec_combine_norm_body, 1.0 / n),
        grid=(nblk,),
        in_specs=specs + stat_spec,
        out_specs=pl.BlockSpec((nb_, co), lambda i: (i, 0)),
        out_shape=jax.ShapeDtypeStruct((n, co), jnp.float32),
    )(*args, sums, ssqs)


def _forward(x0, P, geom, interp, offs_list):
    x1 = _enc1(x0, P["enc1_W"], P["enc1_g"], P["enc1_b"])
    feats = [x1]
    cur = x1
    for li in range(2, 6):
        g = geom[li - 2]
        rel = jnp.asarray(g["rel"])
        nbr = jnp.asarray(g["nbr"])
        ns = nbr.shape[1]
        gfeat = _sc_gather(cur, nbr.reshape(-1))
        rel_flat = rel.reshape(-1, 3)
        cur = _enc_level(rel_flat, gfeat, P["enc%d_W" % li],
                         P["enc%d_g" % li], P["enc%d_b" % li], ns)
        feats.append(cur)
    x1, x2, x3, x4, x5 = feats
    up = _dec5(x5, P)
    skips = [x4, x3, x2, x1]
    for di, skip in zip([4, 3, 2, 1], skips):
        ii, ww = interp[di]
        bfeat = _lin_bn_relu(up, P["dec%d_l2_W" % di], P["dec%d_l2_b" % di],
                             P["dec%d_l2_g" % di], P["dec%d_l2_bb" % di])
        gb = _sc_gather(bfeat, ii.reshape(-1)).reshape(ii.shape[0], -1)
        up = _dec_level(skip, P["dec%d_l1_W" % di], P["dec%d_l1_b" % di],
                        P["dec%d_l1_g" % di], P["dec%d_l1_bb" % di],
                        gb, ww)
    return up


def kernel(p, x, o, params):
    nb = o.shape[0]
    seg = p.shape[0] // nb
    geom = _geometry(p, o)
    offs0 = [(b + 1) * seg for b in range(nb)]
    offs_list = [offs0] + [g["offs"] for g in geom]
    p_levels = [p]
    cur = p
    for g in geom:
        cur = cur[g["samp"]]
        p_levels.append(cur)
    interp = {}
    for di, (fi, ci) in zip([4, 3, 2, 1], [(3, 4), (2, 3), (1, 2), (0, 1)]):
        ii, ww = _interp_geom(p_levels[fi], offs_list[fi], p_levels[ci], offs_list[ci])
        interp[di] = (ii, ww)
    x0 = jnp.concatenate([p, x], 1)
    return _forward(x0, params, geom, interp, offs_list)


# speedup vs baseline: 1.0298x; 1.0106x over previous
import functools

import jax
import jax.numpy as jnp
from jax import lax
from jax.experimental import pallas as pl
from jax.experimental.pallas import tpu as pltpu
from jax.experimental.pallas import tpu_sc as plsc

_STRIDES = [4, 4, 4, 4]
_NSAMPLE = [16, 16, 16, 16]


def _fps_body(pxyz_ref, out_ref):
    nb, n = pxyz_ref.shape[1], pxyz_ref.shape[2]
    m = out_ref.shape[0]
    px = pxyz_ref[0]
    py = pxyz_ref[1]
    pz = pxyz_ref[2]
    bw = min(n, 128)
    nblk = n // bw
    lane = jax.lax.broadcasted_iota(jnp.int32, (nb, bw), 1)
    out_ref[0:1] = jnp.zeros((1, nb, 8), jnp.int32)

    def body(i, carry):
        dist, selx, sely, selz = carry
        dx = px - selx
        dy = py - sely
        dz = pz - selz
        d = dx * dx + dy * dy + dz * dz
        dist = jnp.minimum(dist, d)
        best = dist[:, 0:bw]
        bidx = lane
        for b in range(1, nblk):
            c = dist[:, b * bw:(b + 1) * bw]
            take = c > best
            best = jnp.where(take, c, best)
            bidx = jnp.where(take, lane + b * bw, bidx)
        mx = jnp.max(best, axis=1, keepdims=True)
        idx = jnp.min(jnp.where(best == mx, bidx, n), axis=1, keepdims=True)
        out_ref[pl.ds(i, 1)] = jnp.broadcast_to(idx, (nb, 8))[None]
        accx = jnp.zeros((nb, bw), jnp.float32)
        accy = accx
        accz = accx
        for b in range(nblk):
            sel = (lane + b * bw) == idx
            accx = accx + jnp.where(sel, px[:, b * bw:(b + 1) * bw], 0.0)
            accy = accy + jnp.where(sel, py[:, b * bw:(b + 1) * bw], 0.0)
            accz = accz + jnp.where(sel, pz[:, b * bw:(b + 1) * bw], 0.0)
        selx = jnp.sum(accx, axis=1, keepdims=True)
        sely = jnp.sum(accy, axis=1, keepdims=True)
        selz = jnp.sum(accz, axis=1, keepdims=True)
        return dist, selx, sely, selz

    dist0 = jnp.full((nb, n), jnp.inf, dtype=jnp.float32)
    jax.lax.fori_loop(
        1, m, body, (dist0, px[:, 0:1], py[:, 0:1], pz[:, 0:1]))


def _fps_batched(pts, m):
    nb, n, _ = pts.shape
    pxyz = pts.transpose(2, 0, 1)
    out = pl.pallas_call(
        _fps_body,
        out_shape=jax.ShapeDtypeStruct((m, nb, 8), jnp.int32),
    )(pxyz)
    return out[:, :, 0].transpose(1, 0)


def _topk_body(k, with_w, qx, qy, qz, rx, ry, rz, *outs):
    m = qx.shape[1]
    n = rx.shape[2]
    dx = qx[0] - rx[0]
    dy = qy[0] - ry[0]
    dz = qz[0] - rz[0]
    d = dx * dx + dy * dy + dz * dz
    iota = jax.lax.broadcasted_iota(jnp.int32, (m, n), 1)
    bw = min(n, 128)
    nblk = n // bw
    lane = jax.lax.broadcasted_iota(jnp.int32, (m, bw), 1)
    cols = []
    dds = []
    for _ in range(k):
        best = d[:, 0:bw]
        bidx = lane
        for b in range(1, nblk):
            c = d[:, b * bw:(b + 1) * bw]
            take = c < best
            best = jnp.where(take, c, best)
            bidx = jnp.where(take, lane + b * bw, bidx)
        mn = jnp.min(best, axis=1, keepdims=True)
        idx = jnp.min(jnp.where(best == mn, bidx, n), axis=1, keepdims=True)
        cols.append(idx)
        dds.append(mn)
        d = jnp.where(iota == idx, jnp.inf, d)
    ki = jnp.concatenate(cols, axis=1)
    outs[0][0] = ki
    if with_w:
        kd = jnp.concatenate(dds, axis=1)
        dist = jnp.sqrt(jnp.maximum(kd, 0.0))
        ww = 1.0 / (dist + 1e-8)
        ww = ww / ww.sum(1, keepdims=True)
        outs[1][0] = ww


def _knn_batched(q, ref, k, with_w=False):
    nb, m, _ = q.shape
    n = ref.shape[1]
    qt = q.transpose(2, 0, 1)[..., None]
    rt = ref.transpose(2, 0, 1)[:, :, None, :]
    out_shape = [jax.ShapeDtypeStruct((nb, m, k), jnp.int32)]
    out_specs = [pl.BlockSpec((1, m, k), lambda b: (b, 0, 0))]
    if with_w:
        out_shape.append(jax.ShapeDtypeStruct((nb, m, k), jnp.float32))
        out_specs.append(pl.BlockSpec((1, m, k), lambda b: (b, 0, 0)))
    res = pl.pallas_call(
        functools.partial(_topk_body, k, with_w),
        grid=(nb,),
        in_specs=[pl.BlockSpec((1, m, 1), lambda b: (b, 0, 0))] * 3
        + [pl.BlockSpec((1, 1, n), lambda b: (b, 0, 0))] * 3,
        out_specs=out_specs,
        out_shape=out_shape,
    )(qt[0], qt[1], qt[2], rt[0], rt[1], rt[2])
    return res if with_w else res[0]


def _geometry(p0, o):
    nb = o.shape[0]
    seg = p0.shape[0] // nb
    levels = []
    cur_p = p0
    cur_n = seg
    cur_starts = (o - seg).astype(jnp.int32)
    for st, ns in zip(_STRIDES, _NSAMPLE):
        m = cur_n // st
        segs = cur_p.reshape(nb, cur_n, 3)
        fi = _fps_batched(segs, m)
        samp = (fi + cur_starts[:, None]).reshape(-1)
        q = jnp.take_along_axis(segs, fi[..., None], axis=1)
        ki = _knn_batched(q, segs, ns)
        nbr = (ki + cur_starts[:, None, None]).reshape(-1, ns)
        new_p = cur_p[samp]
        rel = cur_p[nbr] - new_p[:, None, :]
        new_offs = [(b + 1) * m for b in range(nb)]
        levels.append({"samp": samp, "nbr": nbr, "rel": rel.astype(jnp.float32), "offs": new_offs})
        cur_p = new_p
        cur_n = m
        cur_starts = jnp.arange(nb, dtype=jnp.int32) * m
    return levels


def _interp_geom(p_fine, offs_fine, p_coarse, offs_coarse):
    nb = len(offs_fine)
    mf = offs_fine[0]
    nc = offs_coarse[0]
    q = p_fine.reshape(nb, mf, 3)
    ref = p_coarse.reshape(nb, nc, 3)
    ki, ww = _knn_batched(q, ref, 3, with_w=True)
    starts = jnp.arange(nb, dtype=jnp.int32)[:, None, None] * nc
    ii = (ki + starts).reshape(-1, 3)
    return ii, ww.reshape(-1, 3)


def _sc_gather(table, idx):
    V, D = table.shape
    B = idx.shape[0]
    NW = 32
    b_per_w = B // NW
    if D % 128 or B % (8 * NW):
        return table[idx]

    mesh = plsc.VectorSubcoreMesh(core_axis_name="c", subcore_axis_name="s")

    @functools.partial(
        pl.kernel,
        mesh=mesh,
        out_type=jax.ShapeDtypeStruct((B, D), jnp.float32),
        scratch_types=[
            pltpu.VMEM((b_per_w,), jnp.int32),
            pltpu.VMEM((b_per_w, D), jnp.float32),
            pltpu.SemaphoreType.DMA,
        ],
    )
    def k(table_hbm, idx_hbm, out_hbm, idx_v, rows_v, sem):
        wid = lax.axis_index("s") * 2 + lax.axis_index("c")
        base = wid * b_per_w
        pltpu.sync_copy(idx_hbm.at[pl.ds(base, b_per_w)], idx_v)
        pltpu.async_copy(table_hbm.at[idx_v], rows_v, sem).wait()
        pltpu.sync_copy(rows_v, out_hbm.at[pl.ds(base, b_per_w)])

    return k(table, idx)


def _bn(x, g, b):
    ax = tuple(range(x.ndim - 1))
    m = x.mean(ax)
    v = x.var(ax)
    return (x - m) / jnp.sqrt(v + 1e-5) * g + b


def _enc1_kernel(x0_ref, w_ref, g_ref, b_ref, out_ref):
    h = jnp.dot(x0_ref[...], w_ref[...], preferred_element_type=jnp.float32)
    m = h.mean(axis=0, keepdims=True)
    v = ((h - m) ** 2).mean(axis=0, keepdims=True)
    hn = (h - m) / jnp.sqrt(v + 1e-5) * g_ref[...] + b_ref[...]
    out_ref[...] = jnp.maximum(hn, 0.0)


def _enc1(x0, W, g, b):
    n = x0.shape[0]
    co = W.shape[1]
    return pl.pallas_call(
        _enc1_kernel,
        out_shape=jax.ShapeDtypeStruct((n, co), jnp.float32),
    )(x0, W, g.reshape(1, co), b.reshape(1, co))


def _enc_stats_body(rel_ref, gf_ref, wr_ref, wf_ref, sum_ref, ssq_ref):
    h = jnp.dot(rel_ref[...], wr_ref[...], preferred_element_type=jnp.float32)
    h = h + jnp.dot(gf_ref[...], wf_ref[...], preferred_element_type=jnp.float32)
    s = h.sum(axis=0, keepdims=True)
    ss = (h * h).sum(axis=0, keepdims=True)

    @pl.when(pl.program_id(0) == 0)
    def _init():
        sum_ref[...] = s
        ssq_ref[...] = ss

    @pl.when(pl.program_id(0) > 0)
    def _acc():
        sum_ref[...] += s
        ssq_ref[...] += ss


def _enc_norm_body(nn, binv, rel_ref, gf_ref, wr_ref, wf_ref, g_ref, b_ref,
                   sum_ref, ssq_ref, out_ref):
    h = jnp.dot(rel_ref[...], wr_ref[...], preferred_element_type=jnp.float32)
    h = h + jnp.dot(gf_ref[...], wf_ref[...], preferred_element_type=jnp.float32)
    mu = sum_ref[...] * binv
    var = ssq_ref[...] * binv - mu * mu
    y = (h - mu) / jnp.sqrt(var + 1e-5) * g_ref[...] + b_ref[...]
    y = jnp.maximum(y, 0.0)
    mb = y.shape[0] // nn
    out_ref[...] = y.reshape(mb, nn, y.shape[1]).max(axis=1)


def _enc_single_body(nn, rel_ref, gf_ref, wr_ref, wf_ref, g_ref, b_ref, out_ref):
    h = jnp.dot(rel_ref[...], wr_ref[...], preferred_element_type=jnp.float32)
    h = h + jnp.dot(gf_ref[...], wf_ref[...], preferred_element_type=jnp.float32)
    mu = h.mean(axis=0, keepdims=True)
    var = ((h - mu) ** 2).mean(axis=0, keepdims=True)
    y = (h - mu) / jnp.sqrt(var + 1e-5) * g_ref[...] + b_ref[...]
    y = jnp.maximum(y, 0.0)
    mb = y.shape[0] // nn
    out_ref[...] = y.reshape(mb, nn, y.shape[1]).max(axis=1)


def _enc_level(rel_flat, gfeat, W, g, b, nn):
    B, C = gfeat.shape
    Co = W.shape[1]
    Wr = W[:3]
    Wf = W[3:]
    m = B // nn
    nblk = max(1, B // 8192)
    Bb = B // nblk
    mb = m // nblk
    if nblk == 1:
        return pl.pallas_call(
            functools.partial(_enc_single_body, nn),
            out_shape=jax.ShapeDtypeStruct((m, Co), jnp.float32),
        )(rel_flat, gfeat, Wr, Wf, g.reshape(1, Co), b.reshape(1, Co))
    sums, ssqs = pl.pallas_call(
        _enc_stats_body,
        grid=(nblk,),
        in_specs=[
            pl.BlockSpec((Bb, 3), lambda i: (i, 0)),
            pl.BlockSpec((Bb, C), lambda i: (i, 0)),
            pl.BlockSpec((3, Co), lambda i: (0, 0)),
            pl.BlockSpec((C, Co), lambda i: (0, 0)),
        ],
        out_specs=[
            pl.BlockSpec((1, Co), lambda i: (0, 0)),
            pl.BlockSpec((1, Co), lambda i: (0, 0)),
        ],
        out_shape=[
            jax.ShapeDtypeStruct((1, Co), jnp.float32),
            jax.ShapeDtypeStruct((1, Co), jnp.float32),
        ],
    )(rel_flat, gfeat, Wr, Wf)
    out = pl.pallas_call(
        functools.partial(_enc_norm_body, nn, 1.0 / B),
        grid=(nblk,),
        in_specs=[
            pl.BlockSpec((Bb, 3), lambda i: (i, 0)),
            pl.BlockSpec((Bb, C), lambda i: (i, 0)),
            pl.BlockSpec((3, Co), lambda i: (0, 0)),
            pl.BlockSpec((C, Co), lambda i: (0, 0)),
            pl.BlockSpec((1, Co), lambda i: (0, 0)),
            pl.BlockSpec((1, Co), lambda i: (0, 0)),
            pl.BlockSpec((1, Co), lambda i: (0, 0)),
            pl.BlockSpec((1, Co), lambda i: (0, 0)),
        ],
        out_specs=pl.BlockSpec((mb, Co), lambda i: (i, 0)),
        out_shape=jax.ShapeDtypeStruct((m, Co), jnp.float32),
    )(rel_flat, gfeat, Wr, Wf, g.reshape(1, Co), b.reshape(1, Co), sums, ssqs)
    return out


def _dec5_body(x5_ref, w2_ref, b2_ref, w1_ref, b1_ref, g_ref, bb_ref, out_ref):
    x5 = x5_ref[...]
    n, c = x5.shape
    x3d = x5.reshape(8, n // 8, c)
    mean = x3d.mean(axis=1)
    gf = jnp.dot(mean, w2_ref[...], preferred_element_type=jnp.float32) + b2_ref[...]
    gf = jnp.maximum(gf, 0.0)
    gfb = jnp.broadcast_to(gf[:, None, :], (8, n // 8, gf.shape[1]))
    xc = jnp.concatenate([x3d, gfb], axis=2).reshape(n, c + gf.shape[1])
    h = jnp.dot(xc, w1_ref[...], preferred_element_type=jnp.float32) + b1_ref[...]
    mu = h.mean(axis=0, keepdims=True)
    var = ((h - mu) ** 2).mean(axis=0, keepdims=True)
    y = (h - mu) / jnp.sqrt(var + 1e-5) * g_ref[...] + bb_ref[...]
    out_ref[...] = jnp.maximum(y, 0.0)


def _dec5(x5, P):
    n, c = x5.shape
    co = P["dec5_l1_W"].shape[1]
    return pl.pallas_call(
        _dec5_body,
        out_shape=jax.ShapeDtypeStruct((n, co), jnp.float32),
    )(x5, P["dec5_l2_W"], P["dec5_l2_b"].reshape(1, -1), P["dec5_l1_W"],
      P["dec5_l1_b"].reshape(1, -1), P["dec5_l1_g"].reshape(1, -1),
      P["dec5_l1_bb"].reshape(1, -1))


def _lin_bn_relu_body(x_ref, w_ref, b_ref, g_ref, bb_ref, out_ref):
    h = jnp.dot(x_ref[...], w_ref[...], preferred_element_type=jnp.float32) + b_ref[...]
    mu = h.mean(axis=0, keepdims=True)
    var = ((h - mu) ** 2).mean(axis=0, keepdims=True)
    y = (h - mu) / jnp.sqrt(var + 1e-5) * g_ref[...] + bb_ref[...]
    out_ref[...] = jnp.maximum(y, 0.0)


def _lin_bn_relu(xin, W, b, g, bb):
    n = xin.shape[0]
    co = W.shape[1]
    return pl.pallas_call(
        _lin_bn_relu_body,
        out_shape=jax.ShapeDtypeStruct((n, co), jnp.float32),
    )(xin, W, b.reshape(1, co), g.reshape(1, co), bb.reshape(1, co))


def _dec_combine_body(x_ref, w_ref, b_ref, sum_ref, ssq_ref):
    h = jnp.dot(x_ref[...], w_ref[...], preferred_element_type=jnp.float32) + b_ref[...]
    s = h.sum(axis=0, keepdims=True)
    ss = (h * h).sum(axis=0, keepdims=True)

    @pl.when(pl.program_id(0) == 0)
    def _init():
        sum_ref[...] = s
        ssq_ref[...] = ss

    @pl.when(pl.program_id(0) > 0)
    def _acc():
        sum_ref[...] += s
        ssq_ref[...] += ss


def _dec_combine_norm_body(binv, x_ref, w_ref, b_ref, g_ref, bb_ref, gb_ref,
                           ww_ref, sum_ref, ssq_ref, out_ref):
    h = jnp.dot(x_ref[...], w_ref[...], preferred_element_type=jnp.float32) + b_ref[...]
    mu = sum_ref[...] * binv
    var = ssq_ref[...] * binv - mu * mu
    a = (h - mu) / jnp.sqrt(var + 1e-5) * g_ref[...] + bb_ref[...]
    a = jnp.maximum(a, 0.0)
    co = h.shape[1]
    gb = gb_ref[...]
    ww = ww_ref[...]
    wsum = (gb[:, :co] * ww[:, 0:1] + gb[:, co:2 * co] * ww[:, 1:2]
            + gb[:, 2 * co:] * ww[:, 2:3])
    out_ref[...] = a + wsum


def _dec_single_body(x_ref, w_ref, b_ref, g_ref, bb_ref, gb_ref, ww_ref, out_ref):
    h = jnp.dot(x_ref[...], w_ref[...], preferred_element_type=jnp.float32) + b_ref[...]
    mu = h.mean(axis=0, keepdims=True)
    var = ((h - mu) ** 2).mean(axis=0, keepdims=True)
    a = (h - mu) / jnp.sqrt(var + 1e-5) * g_ref[...] + bb_ref[...]
    a = jnp.maximum(a, 0.0)
    co = h.shape[1]
    gb = gb_ref[...]
    ww = ww_ref[...]
    wsum = (gb[:, :co] * ww[:, 0:1] + gb[:, co:2 * co] * ww[:, 1:2]
            + gb[:, 2 * co:] * ww[:, 2:3])
    out_ref[...] = a + wsum


def _dec_level(skip, W1, b1, g1, bb1, gb, ww):
    n, ci = skip.shape
    co = W1.shape[1]
    nblk = max(1, n // 8192)
    nb_ = n // nblk
    if nblk == 1:
        return pl.pallas_call(
            _dec_single_body,
            out_shape=jax.ShapeDtypeStruct((n, co), jnp.float32),
        )(skip, W1, b1.reshape(1, co), g1.reshape(1, co), bb1.reshape(1, co),
          gb, ww)
    specs = [
        pl.BlockSpec((nb_, ci), lambda i: (i, 0)),
        pl.BlockSpec((ci, co), lambda i: (0, 0)),
        pl.BlockSpec((1, co), lambda i: (0, 0)),
        pl.BlockSpec((1, co), lambda i: (0, 0)),
        pl.BlockSpec((1, co), lambda i: (0, 0)),
        pl.BlockSpec((nb_, 3 * co), lambda i: (i, 0)),
        pl.BlockSpec((nb_, 3), lambda i: (i, 0)),
    ]
    stat_spec = [
        pl.BlockSpec((1, co), lambda i: (0, 0)),
        pl.BlockSpec((1, co), lambda i: (0, 0)),
    ]
    args = (skip, W1, b1.reshape(1, co), g1.reshape(1, co), bb1.reshape(1, co),
            gb, ww)
    sums, ssqs = pl.pallas_call(
        _dec_combine_body,
        grid=(nblk,),
        in_specs=specs[:3],
        out_specs=stat_spec,
        out_shape=[jax.ShapeDtypeStruct((1, co), jnp.float32)] * 2,
    )(*args[:3])
    return pl.pallas_call(
        functools.partial(_dec_combine_norm_body, 1.0 / n),
        grid=(nblk,),
        in_specs=specs + stat_spec,
        out_specs=pl.BlockSpec((nb_, co), lambda i: (i, 0)),
        out_shape=jax.ShapeDtypeStruct((n, co), jnp.float32),
    )(*args, sums, ssqs)


def _forward(x0, P, geom, interp, offs_list):
    x1 = _enc1(x0, P["enc1_W"], P["enc1_g"], P["enc1_b"])
    feats = [x1]
    cur = x1
    for li in range(2, 6):
        g = geom[li - 2]
        rel = jnp.asarray(g["rel"])
        nbr = jnp.asarray(g["nbr"])
        ns = nbr.shape[1]
        gfeat = _sc_gather(cur, nbr.reshape(-1))
        rel_flat = rel.reshape(-1, 3)
        cur = _enc_level(rel_flat, gfeat, P["enc%d_W" % li],
                         P["enc%d_g" % li], P["enc%d_b" % li], ns)
        feats.append(cur)
    x1, x2, x3, x4, x5 = feats
    up = _dec5(x5, P)
    skips = [x4, x3, x2, x1]
    for di, skip in zip([4, 3, 2, 1], skips):
        ii, ww = interp[di]
        bfeat = _lin_bn_relu(up, P["dec%d_l2_W" % di], P["dec%d_l2_b" % di],
                             P["dec%d_l2_g" % di], P["dec%d_l2_bb" % di])
        gb = _sc_gather(bfeat, ii.reshape(-1)).reshape(ii.shape[0], -1)
        up = _dec_level(skip, P["dec%d_l1_W" % di], P["dec%d_l1_b" % di],
                        P["dec%d_l1_g" % di], P["dec%d_l1_bb" % di],
                        gb, ww)
    return up


def kernel(p, x, o, params):
    nb = o.shape[0]
    seg = p.shape[0] // nb
    geom = _geometry(p, o)
    offs0 = [(b + 1) * seg for b in range(nb)]
    offs_list = [offs0] + [g["offs"] for g in geom]
    p_levels = [p]
    cur = p
    for g in geom:
        cur = cur[g["samp"]]
        p_levels.append(cur)
    interp = {}
    for di, (fi, ci) in zip([4, 3, 2, 1], [(3, 4), (2, 3), (1, 2), (0, 1)]):
        ii, ww = _interp_geom(p_levels[fi], offs_list[fi], p_levels[ci], offs_list[ci])
        interp[di] = (ii, ww)
    x0 = jnp.concatenate([p, x], 1)
    return _forward(x0, params, geom, interp, offs_list)


# speedup vs baseline: 1.0310x; 1.0012x over previous
import functools

import jax
import jax.numpy as jnp
from jax import lax
from jax.experimental import pallas as pl
from jax.experimental.pallas import tpu as pltpu
from jax.experimental.pallas import tpu_sc as plsc

_STRIDES = [4, 4, 4, 4]
_NSAMPLE = [16, 16, 16, 16]


def _fps_body(pxyz_ref, out_ref):
    nb, n = pxyz_ref.shape[1], pxyz_ref.shape[2]
    m = out_ref.shape[0]
    px = pxyz_ref[0]
    py = pxyz_ref[1]
    pz = pxyz_ref[2]
    bw = min(n, 128)
    nblk = n // bw
    lane = jax.lax.broadcasted_iota(jnp.int32, (nb, bw), 1)
    out_ref[0:1] = jnp.zeros((1, nb, 8), jnp.int32)

    def body(i, carry):
        dist, selx, sely, selz = carry
        dx = px - selx
        dy = py - sely
        dz = pz - selz
        d = dx * dx + dy * dy + dz * dz
        dist = jnp.minimum(dist, d)
        best = dist[:, 0:bw]
        bidx = lane
        for b in range(1, nblk):
            c = dist[:, b * bw:(b + 1) * bw]
            take = c > best
            best = jnp.where(take, c, best)
            bidx = jnp.where(take, lane + b * bw, bidx)
        mx = jnp.max(best, axis=1, keepdims=True)
        idx = jnp.min(jnp.where(best == mx, bidx, n), axis=1, keepdims=True)
        out_ref[pl.ds(i, 1)] = jnp.broadcast_to(idx, (nb, 8))[None]
        accx = jnp.zeros((nb, bw), jnp.float32)
        accy = accx
        accz = accx
        for b in range(nblk):
            sel = (lane + b * bw) == idx
            accx = accx + jnp.where(sel, px[:, b * bw:(b + 1) * bw], 0.0)
            accy = accy + jnp.where(sel, py[:, b * bw:(b + 1) * bw], 0.0)
            accz = accz + jnp.where(sel, pz[:, b * bw:(b + 1) * bw], 0.0)
        selx = jnp.sum(accx, axis=1, keepdims=True)
        sely = jnp.sum(accy, axis=1, keepdims=True)
        selz = jnp.sum(accz, axis=1, keepdims=True)
        return dist, selx, sely, selz

    dist0 = jnp.full((nb, n), jnp.inf, dtype=jnp.float32)
    jax.lax.fori_loop(
        1, m, body, (dist0, px[:, 0:1], py[:, 0:1], pz[:, 0:1]))


def _fps_batched(pts, m):
    nb, n, _ = pts.shape
    pxyz = pts.transpose(2, 0, 1)
    out = pl.pallas_call(
        _fps_body,
        out_shape=jax.ShapeDtypeStruct((m, nb, 8), jnp.int32),
    )(pxyz)
    return out[:, :, 0].transpose(1, 0)


def _topk_body(k, with_w, qx, qy, qz, rx, ry, rz, *outs):
    m = qx.shape[1]
    n = rx.shape[2]
    dx = qx[0] - rx[0]
    dy = qy[0] - ry[0]
    dz = qz[0] - rz[0]
    d = dx * dx + dy * dy + dz * dz
    iota = jax.lax.broadcasted_iota(jnp.int32, (m, n), 1)
    bw = min(n, 128)
    nblk = n // bw
    lane = jax.lax.broadcasted_iota(jnp.int32, (m, bw), 1)
    cols = []
    dds = []
    for _ in range(k):
        best = d[:, 0:bw]
        bidx = lane
        for b in range(1, nblk):
            c = d[:, b * bw:(b + 1) * bw]
            take = c < best
            best = jnp.where(take, c, best)
            bidx = jnp.where(take, lane + b * bw, bidx)
        mn = jnp.min(best, axis=1, keepdims=True)
        idx = jnp.min(jnp.where(best == mn, bidx, n), axis=1, keepdims=True)
        cols.append(idx)
        dds.append(mn)
        d = jnp.where(iota == idx, jnp.inf, d)
    ki = jnp.concatenate(cols, axis=1)
    outs[0][0] = ki
    if with_w:
        kd = jnp.concatenate(dds, axis=1)
        dist = jnp.sqrt(jnp.maximum(kd, 0.0))
        ww = 1.0 / (dist + 1e-8)
        ww = ww / ww.sum(1, keepdims=True)
        outs[1][0] = ww


def _knn_batched(q, ref, k, with_w=False):
    nb, m, _ = q.shape
    n = ref.shape[1]
    qt = q.transpose(2, 0, 1)[..., None]
    rt = ref.transpose(2, 0, 1)[:, :, None, :]
    out_shape = [jax.ShapeDtypeStruct((nb, m, k), jnp.int32)]
    out_specs = [pl.BlockSpec((1, m, k), lambda b: (b, 0, 0))]
    if with_w:
        out_shape.append(jax.ShapeDtypeStruct((nb, m, k), jnp.float32))
        out_specs.append(pl.BlockSpec((1, m, k), lambda b: (b, 0, 0)))
    res = pl.pallas_call(
        functools.partial(_topk_body, k, with_w),
        grid=(nb,),
        in_specs=[pl.BlockSpec((1, m, 1), lambda b: (b, 0, 0))] * 3
        + [pl.BlockSpec((1, 1, n), lambda b: (b, 0, 0))] * 3,
        out_specs=out_specs,
        out_shape=out_shape,
    )(qt[0], qt[1], qt[2], rt[0], rt[1], rt[2])
    return res if with_w else res[0]


def _geometry(p0, o):
    nb = o.shape[0]
    seg = p0.shape[0] // nb
    levels = []
    cur_p = p0
    cur_n = seg
    cur_starts = (o - seg).astype(jnp.int32)
    for st, ns in zip(_STRIDES, _NSAMPLE):
        m = cur_n // st
        segs = cur_p.reshape(nb, cur_n, 3)
        fi = _fps_batched(segs, m)
        samp = (fi + cur_starts[:, None]).reshape(-1)
        q = jnp.take_along_axis(segs, fi[..., None], axis=1)
        ki = _knn_batched(q, segs, ns)
        nbr = (ki + cur_starts[:, None, None]).reshape(-1, ns)
        new_p = cur_p[samp]
        rel = cur_p[nbr] - new_p[:, None, :]
        new_offs = [(b + 1) * m for b in range(nb)]
        levels.append({"samp": samp, "nbr": nbr, "rel": rel.astype(jnp.float32), "offs": new_offs})
        cur_p = new_p
        cur_n = m
        cur_starts = jnp.arange(nb, dtype=jnp.int32) * m
    return levels


def _interp_geom(p_fine, offs_fine, p_coarse, offs_coarse):
    nb = len(offs_fine)
    mf = offs_fine[0]
    nc = offs_coarse[0]
    q = p_fine.reshape(nb, mf, 3)
    ref = p_coarse.reshape(nb, nc, 3)
    ki, ww = _knn_batched(q, ref, 3, with_w=True)
    starts = jnp.arange(nb, dtype=jnp.int32)[:, None, None] * nc
    ii = (ki + starts).reshape(-1, 3)
    return ii, ww.reshape(-1, 3)


def _sc_gather(table, idx):
    V, D = table.shape
    B = idx.shape[0]
    NW = 32
    b_per_w = B // NW
    if D % 128 or B % (8 * NW):
        return table[idx]

    mesh = plsc.VectorSubcoreMesh(core_axis_name="c", subcore_axis_name="s")

    @functools.partial(
        pl.kernel,
        mesh=mesh,
        out_type=jax.ShapeDtypeStruct((B, D), jnp.float32),
        scratch_types=[
            pltpu.VMEM((b_per_w,), jnp.int32),
            pltpu.VMEM((b_per_w, D), jnp.float32),
            pltpu.SemaphoreType.DMA,
        ],
    )
    def k(table_hbm, idx_hbm, out_hbm, idx_v, rows_v, sem):
        wid = lax.axis_index("s") * 2 + lax.axis_index("c")
        base = wid * b_per_w
        pltpu.sync_copy(idx_hbm.at[pl.ds(base, b_per_w)], idx_v)
        pltpu.async_copy(table_hbm.at[idx_v], rows_v, sem).wait()
        pltpu.sync_copy(rows_v, out_hbm.at[pl.ds(base, b_per_w)])

    return k(table, idx)


def _enc1_kernel(x0_ref, w_ref, g_ref, b_ref, out_ref):
    h = jnp.dot(x0_ref[...], w_ref[...], preferred_element_type=jnp.float32)
    m = h.mean(axis=0, keepdims=True)
    v = ((h - m) ** 2).mean(axis=0, keepdims=True)
    hn = (h - m) / jnp.sqrt(v + 1e-5) * g_ref[...] + b_ref[...]
    out_ref[...] = jnp.maximum(hn, 0.0)


def _enc1(x0, W, g, b):
    n = x0.shape[0]
    co = W.shape[1]
    return pl.pallas_call(
        _enc1_kernel,
        out_shape=jax.ShapeDtypeStruct((n, co), jnp.float32),
    )(x0, W, g.reshape(1, co), b.reshape(1, co))


def _enc_stats_body(rel_ref, gf_ref, wr_ref, wf_ref, sum_ref, ssq_ref):
    h = jnp.dot(rel_ref[...], wr_ref[...], preferred_element_type=jnp.float32)
    h = h + jnp.dot(gf_ref[...], wf_ref[...], preferred_element_type=jnp.float32)
    s = h.sum(axis=0, keepdims=True)
    ss = (h * h).sum(axis=0, keepdims=True)

    @pl.when(pl.program_id(0) == 0)
    def _init():
        sum_ref[...] = s
        ssq_ref[...] = ss

    @pl.when(pl.program_id(0) > 0)
    def _acc():
        sum_ref[...] += s
        ssq_ref[...] += ss


def _enc_norm_body(nn, binv, rel_ref, gf_ref, wr_ref, wf_ref, g_ref, b_ref,
                   sum_ref, ssq_ref, out_ref):
    h = jnp.dot(rel_ref[...], wr_ref[...], preferred_element_type=jnp.float32)
    h = h + jnp.dot(gf_ref[...], wf_ref[...], preferred_element_type=jnp.float32)
    mu = sum_ref[...] * binv
    var = ssq_ref[...] * binv - mu * mu
    y = (h - mu) / jnp.sqrt(var + 1e-5) * g_ref[...] + b_ref[...]
    y = jnp.maximum(y, 0.0)
    mb = y.shape[0] // nn
    out_ref[...] = y.reshape(mb, nn, y.shape[1]).max(axis=1)


def _enc_single_body(nn, rel_ref, gf_ref, wr_ref, wf_ref, g_ref, b_ref, out_ref):
    h = jnp.dot(rel_ref[...], wr_ref[...], preferred_element_type=jnp.float32)
    h = h + jnp.dot(gf_ref[...], wf_ref[...], preferred_element_type=jnp.float32)
    mu = h.mean(axis=0, keepdims=True)
    var = ((h - mu) ** 2).mean(axis=0, keepdims=True)
    y = (h - mu) / jnp.sqrt(var + 1e-5) * g_ref[...] + b_ref[...]
    y = jnp.maximum(y, 0.0)
    mb = y.shape[0] // nn
    out_ref[...] = y.reshape(mb, nn, y.shape[1]).max(axis=1)


def _enc_level(rel_flat, gfeat, W, g, b, nn):
    B, C = gfeat.shape
    Co = W.shape[1]
    Wr = W[:3]
    Wf = W[3:]
    m = B // nn
    nblk = max(1, B // 8192)
    Bb = B // nblk
    mb = m // nblk
    if nblk == 1:
        return pl.pallas_call(
            functools.partial(_enc_single_body, nn),
            out_shape=jax.ShapeDtypeStruct((m, Co), jnp.float32),
        )(rel_flat, gfeat, Wr, Wf, g.reshape(1, Co), b.reshape(1, Co))
    sums, ssqs = pl.pallas_call(
        _enc_stats_body,
        grid=(nblk,),
        in_specs=[
            pl.BlockSpec((Bb, 3), lambda i: (i, 0)),
            pl.BlockSpec((Bb, C), lambda i: (i, 0)),
            pl.BlockSpec((3, Co), lambda i: (0, 0)),
            pl.BlockSpec((C, Co), lambda i: (0, 0)),
        ],
        out_specs=[
            pl.BlockSpec((1, Co), lambda i: (0, 0)),
            pl.BlockSpec((1, Co), lambda i: (0, 0)),
        ],
        out_shape=[
            jax.ShapeDtypeStruct((1, Co), jnp.float32),
            jax.ShapeDtypeStruct((1, Co), jnp.float32),
        ],
    )(rel_flat, gfeat, Wr, Wf)
    out = pl.pallas_call(
        functools.partial(_enc_norm_body, nn, 1.0 / B),
        grid=(nblk,),
        in_specs=[
            pl.BlockSpec((Bb, 3), lambda i: (i, 0)),
            pl.BlockSpec((Bb, C), lambda i: (i, 0)),
            pl.BlockSpec((3, Co), lambda i: (0, 0)),
            pl.BlockSpec((C, Co), lambda i: (0, 0)),
            pl.BlockSpec((1, Co), lambda i: (0, 0)),
            pl.BlockSpec((1, Co), lambda i: (0, 0)),
            pl.BlockSpec((1, Co), lambda i: (0, 0)),
            pl.BlockSpec((1, Co), lambda i: (0, 0)),
        ],
        out_specs=pl.BlockSpec((mb, Co), lambda i: (i, 0)),
        out_shape=jax.ShapeDtypeStruct((m, Co), jnp.float32),
    )(rel_flat, gfeat, Wr, Wf, g.reshape(1, Co), b.reshape(1, Co), sums, ssqs)
    return out


def _dec5_body(x5_ref, w2_ref, b2_ref, w1_ref, b1_ref, g_ref, bb_ref, out_ref):
    x5 = x5_ref[...]
    n, c = x5.shape
    x3d = x5.reshape(8, n // 8, c)
    mean = x3d.mean(axis=1)
    gf = jnp.dot(mean, w2_ref[...], preferred_element_type=jnp.float32) + b2_ref[...]
    gf = jnp.maximum(gf, 0.0)
    gfb = jnp.broadcast_to(gf[:, None, :], (8, n // 8, gf.shape[1]))
    xc = jnp.concatenate([x3d, gfb], axis=2).reshape(n, c + gf.shape[1])
    h = jnp.dot(xc, w1_ref[...], preferred_element_type=jnp.float32) + b1_ref[...]
    mu = h.mean(axis=0, keepdims=True)
    var = ((h - mu) ** 2).mean(axis=0, keepdims=True)
    y = (h - mu) / jnp.sqrt(var + 1e-5) * g_ref[...] + bb_ref[...]
    out_ref[...] = jnp.maximum(y, 0.0)


def _dec5(x5, P):
    n, c = x5.shape
    co = P["dec5_l1_W"].shape[1]
    return pl.pallas_call(
        _dec5_body,
        out_shape=jax.ShapeDtypeStruct((n, co), jnp.float32),
    )(x5, P["dec5_l2_W"], P["dec5_l2_b"].reshape(1, -1), P["dec5_l1_W"],
      P["dec5_l1_b"].reshape(1, -1), P["dec5_l1_g"].reshape(1, -1),
      P["dec5_l1_bb"].reshape(1, -1))


def _lin_bn_relu_body(x_ref, w_ref, b_ref, g_ref, bb_ref, out_ref):
    h = jnp.dot(x_ref[...], w_ref[...], preferred_element_type=jnp.float32) + b_ref[...]
    mu = h.mean(axis=0, keepdims=True)
    var = ((h - mu) ** 2).mean(axis=0, keepdims=True)
    y = (h - mu) / jnp.sqrt(var + 1e-5) * g_ref[...] + bb_ref[...]
    out_ref[...] = jnp.maximum(y, 0.0)


def _lin_bn_relu(xin, W, b, g, bb):
    n = xin.shape[0]
    co = W.shape[1]
    return pl.pallas_call(
        _lin_bn_relu_body,
        out_shape=jax.ShapeDtypeStruct((n, co), jnp.float32),
    )(xin, W, b.reshape(1, co), g.reshape(1, co), bb.reshape(1, co))


def _dec_combine_body(x_ref, w_ref, b_ref, sum_ref, ssq_ref):
    h = jnp.dot(x_ref[...], w_ref[...], preferred_element_type=jnp.float32) + b_ref[...]
    s = h.sum(axis=0, keepdims=True)
    ss = (h * h).sum(axis=0, keepdims=True)

    @pl.when(pl.program_id(0) == 0)
    def _init():
        sum_ref[...] = s
        ssq_ref[...] = ss

    @pl.when(pl.program_id(0) > 0)
    def _acc():
        sum_ref[...] += s
        ssq_ref[...] += ss


def _dec_combine_norm_body(binv, x_ref, w_ref, b_ref, g_ref, bb_ref, gb_ref,
                           ww_ref, sum_ref, ssq_ref, out_ref):
    h = jnp.dot(x_ref[...], w_ref[...], preferred_element_type=jnp.float32) + b_ref[...]
    mu = sum_ref[...] * binv
    var = ssq_ref[...] * binv - mu * mu
    a = (h - mu) / jnp.sqrt(var + 1e-5) * g_ref[...] + bb_ref[...]
    a = jnp.maximum(a, 0.0)
    co = h.shape[1]
    gb = gb_ref[...]
    ww = ww_ref[...]
    wsum = (gb[:, :co] * ww[:, 0:1] + gb[:, co:2 * co] * ww[:, 1:2]
            + gb[:, 2 * co:] * ww[:, 2:3])
    out_ref[...] = a + wsum


def _dec_single_body(x_ref, w_ref, b_ref, g_ref, bb_ref, gb_ref, ww_ref, out_ref):
    h = jnp.dot(x_ref[...], w_ref[...], preferred_element_type=jnp.float32) + b_ref[...]
    mu = h.mean(axis=0, keepdims=True)
    var = ((h - mu) ** 2).mean(axis=0, keepdims=True)
    a = (h - mu) / jnp.sqrt(var + 1e-5) * g_ref[...] + bb_ref[...]
    a = jnp.maximum(a, 0.0)
    co = h.shape[1]
    gb = gb_ref[...]
    ww = ww_ref[...]
    wsum = (gb[:, :co] * ww[:, 0:1] + gb[:, co:2 * co] * ww[:, 1:2]
            + gb[:, 2 * co:] * ww[:, 2:3])
    out_ref[...] = a + wsum


def _dec_level(skip, W1, b1, g1, bb1, gb, ww):
    n, ci = skip.shape
    co = W1.shape[1]
    nblk = max(1, n // 8192)
    nb_ = n // nblk
    if nblk == 1:
        return pl.pallas_call(
            _dec_single_body,
            out_shape=jax.ShapeDtypeStruct((n, co), jnp.float32),
        )(skip, W1, b1.reshape(1, co), g1.reshape(1, co), bb1.reshape(1, co),
          gb, ww)
    specs = [
        pl.BlockSpec((nb_, ci), lambda i: (i, 0)),
        pl.BlockSpec((ci, co), lambda i: (0, 0)),
        pl.BlockSpec((1, co), lambda i: (0, 0)),
        pl.BlockSpec((1, co), lambda i: (0, 0)),
        pl.BlockSpec((1, co), lambda i: (0, 0)),
        pl.BlockSpec((nb_, 3 * co), lambda i: (i, 0)),
        pl.BlockSpec((nb_, 3), lambda i: (i, 0)),
    ]
    stat_spec = [
        pl.BlockSpec((1, co), lambda i: (0, 0)),
        pl.BlockSpec((1, co), lambda i: (0, 0)),
    ]
    args = (skip, W1, b1.reshape(1, co), g1.reshape(1, co), bb1.reshape(1, co),
            gb, ww)
    sums, ssqs = pl.pallas_call(
        _dec_combine_body,
        grid=(nblk,),
        in_specs=specs[:3],
        out_specs=stat_spec,
        out_shape=[jax.ShapeDtypeStruct((1, co), jnp.float32)] * 2,
    )(*args[:3])
    return pl.pallas_call(
        functools.partial(_dec_combine_norm_body, 1.0 / n),
        grid=(nblk,),
        in_specs=specs + stat_spec,
        out_specs=pl.BlockSpec((nb_, co), lambda i: (i, 0)),
        out_shape=jax.ShapeDtypeStruct((n, co), jnp.float32),
    )(*args, sums, ssqs)


def _forward(x0, P, geom, interp, offs_list):
    x1 = _enc1(x0, P["enc1_W"], P["enc1_g"], P["enc1_b"])
    feats = [x1]
    cur = x1
    for li in range(2, 6):
        g = geom[li - 2]
        rel = jnp.asarray(g["rel"])
        nbr = jnp.asarray(g["nbr"])
        ns = nbr.shape[1]
        gfeat = _sc_gather(cur, nbr.reshape(-1))
        rel_flat = rel.reshape(-1, 3)
        cur = _enc_level(rel_flat, gfeat, P["enc%d_W" % li],
                         P["enc%d_g" % li], P["enc%d_b" % li], ns)
        feats.append(cur)
    x1, x2, x3, x4, x5 = feats
    up = _dec5(x5, P)
    skips = [x4, x3, x2, x1]
    for di, skip in zip([4, 3, 2, 1], skips):
        ii, ww = interp[di]
        bfeat = _lin_bn_relu(up, P["dec%d_l2_W" % di], P["dec%d_l2_b" % di],
                             P["dec%d_l2_g" % di], P["dec%d_l2_bb" % di])
        gb = _sc_gather(bfeat, ii.reshape(-1)).reshape(ii.shape[0], -1)
        up = _dec_level(skip, P["dec%d_l1_W" % di], P["dec%d_l1_b" % di],
                        P["dec%d_l1_g" % di], P["dec%d_l1_bb" % di],
                        gb, ww)
    return up


def kernel(p, x, o, params):
    nb = o.shape[0]
    seg = p.shape[0] // nb
    geom = _geometry(p, o)
    offs0 = [(b + 1) * seg for b in range(nb)]
    offs_list = [offs0] + [g["offs"] for g in geom]
    p_levels = [p]
    cur = p
    for g in geom:
        cur = cur[g["samp"]]
        p_levels.append(cur)
    interp = {}
    for di, (fi, ci) in zip([4, 3, 2, 1], [(3, 4), (2, 3), (1, 2), (0, 1)]):
        ii, ww = _interp_geom(p_levels[fi], offs_list[fi], p_levels[ci], offs_list[ci])
        interp[di] = (ii, ww)
    x0 = jnp.concatenate([p, x], 1)
    return _forward(x0, params, geom, interp, offs_list)
